# baseline (speedup 1.0000x reference)
import math
import functools

import jax
import jax.numpy as jnp
from jax import lax
from jax.experimental import pallas as pl
from jax.experimental.pallas import tpu as pltpu
from jax.experimental.pallas import tpu_sc as plsc

N = 50000
E = 800000
IN_DIM = 47
H = 64
NRBF = 64
BOND = 13
NB = 2
NGRAPH = 64
OUT = 64
CUT = 10.0

NC, NS = 2, 16
NW = NC * NS
EBLK = 128
BLK_PER_W = 196
E_PAD = NW * BLK_PER_W * EBLK
N_PAD = 51200

BE = 1024
GRID_E = E_PAD // BE
BN = 512
GRID_N = N_PAD // BN

_F32 = jnp.float32


K_G = 7


def _sc_gather(tab, idxs):
    D = tab.shape[1]
    n = len(idxs)
    ngrp = BLK_PER_W // K_G
    mesh = plsc.VectorSubcoreMesh(core_axis_name="c", subcore_axis_name="s",
                                  num_cores=NC, num_subcores=NS)
    out_type = [jax.ShapeDtypeStruct((E_PAD, D), _F32)] * n
    scratch = [pltpu.VMEM((BLK_PER_W, EBLK), jnp.int32),
               pltpu.VMEM((K_G, EBLK, D), _F32),
               pltpu.SemaphoreType.DMA,
               pltpu.SemaphoreType.DMA]

    def body(tab_ref, *rest):
        idx_refs = rest[:n]
        out_refs = rest[n:2 * n]
        idxb, rowsb, sem_g, sem_w = rest[2 * n:]
        c = lax.axis_index("c")
        s = lax.axis_index("s")
        w = s * NC + c
        blk0 = w * BLK_PER_W

        for i in range(n):
            pltpu.sync_copy(idx_refs[i].at[pl.ds(blk0, BLK_PER_W), :], idxb)

            def grp(g, carry):
                b0 = g * K_G
                gd = [pltpu.async_copy(tab_ref.at[idxb.at[b0 + k]],
                                       rowsb.at[k], sem_g)
                      for k in range(K_G)]
                for d in gd:
                    d.wait()
                wd = [pltpu.async_copy(
                    rowsb.at[k],
                    out_refs[i].at[pl.ds((blk0 + b0 + k) * EBLK, EBLK), :],
                    sem_w) for k in range(K_G)]
                for d in wd:
                    d.wait()
                return carry

            lax.fori_loop(0, ngrp, grp, 0)

    f = pl.kernel(body, out_type=out_type, mesh=mesh, scratch_types=scratch,
                  compiler_params=pltpu.CompilerParams(
                      use_tc_tiling_on_sc=False))
    return f(tab, *idxs)


def _sc_scatter(msg, idx, dh, edge_split=False):
    p = msg.shape[0]
    rps = N_PAD // NS
    rb = rps // EBLK
    nblk = E_PAD // EBLK
    if edge_split:
        bps = nblk // (NC * NS)
        ks = 7
    else:
        bps = nblk // NS
        ks = 4 if dh >= 32 else 8
    ngrp = bps // ks
    mesh = plsc.VectorSubcoreMesh(core_axis_name="c", subcore_axis_name="s",
                                  num_cores=NC, num_subcores=NS)
    out_type = jax.ShapeDtypeStruct((2, N_PAD, dh), _F32)
    scratch = [pltpu.VMEM_SHARED((N_PAD, dh), _F32),
               pltpu.VMEM((ks, EBLK), jnp.int32),
               pltpu.VMEM((ks, EBLK, dh), _F32),
               pltpu.VMEM((EBLK, dh), _F32),
               pltpu.SemaphoreType.DMA,
               pltpu.SemaphoreType.DMA]
    offs = [o for o in (0, 16, 24, 32, 48) if o + 16 <= dh]

    def body(msg_ref, idx_ref, out_ref, acc, idxb, msgb, zbuf, sem_m, sem_s):
        c = lax.axis_index("c")
        s = lax.axis_index("s")
        plane = c * (p - 1)

        def zrow(j, carry):
            for o in offs:
                zbuf[j, pl.ds(o, 16)] = jnp.zeros((16,), _F32)
            return carry

        lax.fori_loop(0, EBLK, zrow, 0)

        zd = [pltpu.async_copy(zbuf, acc.at[pl.ds(s * rps + r * EBLK, EBLK),
                                            :], sem_m) for r in range(rb)]
        for d in zd:
            d.wait()
        plsc.subcore_barrier()

        if edge_split:
            myblk0 = (c * NS + s) * bps
        else:
            myblk0 = s * bps

        def grp(g, carry):
            b0 = myblk0 + g * ks
            pltpu.sync_copy(idx_ref.at[pl.ds(b0, ks), :], idxb)
            md = [pltpu.async_copy(
                msg_ref.at[plane, pl.ds((b0 + k) * EBLK, EBLK), :],
                msgb.at[k], sem_m) for k in range(ks)]
            for d in md:
                d.wait()
            sd = [pltpu.async_copy(msgb.at[k], acc.at[idxb.at[k]], sem_s,
                                   add=True) for k in range(ks)]
            for d in sd:
                d.wait()
            return carry

        lax.fori_loop(0, ngrp, grp, 0)
        plsc.subcore_barrier()

        cd = [pltpu.async_copy(acc.at[pl.ds(s * rps + r * EBLK, EBLK), :],
                               out_ref.at[c, pl.ds(s * rps + r * EBLK, EBLK),
                                          :], sem_m) for r in range(rb)]
        for d in cd:
            d.wait()

    f = pl.kernel(body, out_type=out_type, mesh=mesh, scratch_types=scratch,
                  compiler_params=pltpu.CompilerParams(
                      use_tc_tiling_on_sc=False))
    return f(msg, idx)


def _silu(t):
    return t * jax.nn.sigmoid(t)


def _gelu(t):
    return 0.5 * t * (1.0 + lax.erf(t * 0.7071067811865476))


def _lnorm(t, g, b):
    m = jnp.mean(t, -1, keepdims=True)
    v = jnp.mean((t - m) ** 2, -1, keepdims=True)
    return (t - m) * lax.rsqrt(v + 1e-5) * g + b


def _lnorm_mx(t, g, b, mones):
    r = t - t @ mones
    v = (r * r) @ mones
    return r * lax.rsqrt(v + 1e-5) * g + b


def _rbf_tc(d, rw):
    env = 0.5 * (jnp.cos(d * (math.pi / CUT)) + 1.0)
    env = env * jnp.where(d < CUT, 1.0, 0.0)
    r = jnp.exp(-0.5 * ((d - rw["c"]) * rw["inv_w"]) ** 2)
    hh = r * env
    return _silu(hh @ rw["p0w"] + rw["p0b"]) @ rw["p1w"] + rw["p1b"]


def _bspec(shape, emap):
    return pl.BlockSpec(shape, emap)


_EMAP = lambda i: (i, 0)
_WMAP0 = lambda i: (0, 0)
_ACC3 = lambda i: (0, i, 0)

_ARB = pltpu.CompilerParams(dimension_semantics=("arbitrary",))


def _edge_specs(n80, n16, n1):
    specs = [_bspec((BE, 80), _EMAP)] * n80
    specs += [_bspec((BE, 16), _EMAP)] * n16
    specs += [_bspec((BE, 1), _EMAP)] * n1
    return specs


def _wspecs(ws):
    return [_bspec(w.shape, _WMAP0) for w in ws]


def _tc_input(xp, iw, mm64):
    ws = [iw["w"], iw["b"], iw["g"], iw["bb"], mm64]

    def body(x_ref, w_ref, b_ref, g_ref, gb_ref, m64_ref, o_ref):
        t = _gelu(x_ref[...] @ w_ref[...] + b_ref[...])
        o_ref[...] = _lnorm_mx(t, g_ref[...], gb_ref[...], m64_ref[...])

    return pl.pallas_call(
        body, grid=(GRID_N,),
        in_specs=[_bspec((BN, 48), _EMAP)] + _wspecs(ws),
        out_specs=_bspec((BN, 64), _EMAP),
        out_shape=jax.ShapeDtypeStruct((N_PAD, 64), _F32),
        compiler_params=_ARB,
    )(xp, *ws)


def _tc_edge(tr, tc_, ea16, rbf_in, ew, rw, mm128, sp, first):
    ws = [ew["whr"], ew["whc"], ew["wd"], ew["wea"], ew["wrb"], ew["e0b"],
          ew["elng"], ew["elnb"], ew["e1w"], ew["e1b"],
          ew["attw"], ew["attb"], ew["c0w"], ew["c0b"], ew["c1w"], mm128,
          sp["s16"], sp["o16_128"], sp["o8_64"], sp["o8_16"]]
    if first:
        ws += [sp["o16_64"], rw["c"], rw["inv_w"], rw["p0w"], rw["p0b"],
               rw["p1w"], rw["p1b"]]
    nw = len(ws)

    def body(tr_ref, tc_ref, ea_ref, *rest):
        if first:
            wrefs = rest[:nw]
            mm_ref, pd_ref, rb_ref = rest[nw:]
        else:
            rb_in_ref = rest[0]
            wrefs = rest[1:1 + nw]
            mm_ref, pd_ref = rest[1 + nw:]
        (whr, whc, wd, wea, wrb, e0b, elng, elnb, e1w, e1b,
         attw, attb, c0w, c0b, c1w, m128, s16, o16_128, o8_64, o8_16) = (
            r[...] for r in wrefs[:20])
        trv = tr_ref[...]
        tcv = tc_ref[...]
        diff16 = trv[:, 64:80] - tcv[:, 64:80]
        dist16 = jnp.maximum(
            jnp.sqrt((diff16 * diff16) @ s16), 1e-5)
        if first:
            o16_64, rc, riw, rp0w, rp0b, rp1w, rp1b = (
                r[...] for r in wrefs[20:])
            rwd = {"c": rc, "inv_w": riw, "p0w": rp0w, "p0b": rp0b,
                   "p1w": rp1w, "p1b": rp1b}
            rb = _rbf_tc(dist16 @ o16_64, rwd)
            rb_ref[...] = rb
        else:
            rb = rb_in_ref[...]
        t = (trv @ whr + tcv @ whc + (dist16 @ o16_128) * wd
             + ea_ref[...] @ wea + rb @ wrb + e0b)
        t = _lnorm_mx(_silu(t), elng, elnb, m128)
        m = _silu(t @ e1w + e1b)
        att = jax.nn.sigmoid(m @ attw + attb)
        matt = m * (att @ o8_64)
        cw = _silu(m @ c0w + c0b) @ c1w
        mm_ref[0, :, :] = matt[:, 0:32]
        mm_ref[1, :, :] = matt[:, 32:64]
        pd_ref[0, :, :] = diff16 * (cw @ o8_16)

    out_shape = [jax.ShapeDtypeStruct((2, E_PAD, 32), _F32),
                 jax.ShapeDtypeStruct((1, E_PAD, 16), _F32)]
    out_specs = [pl.BlockSpec((2, BE, 32), _ACC3),
                 pl.BlockSpec((1, BE, 16), _ACC3)]
    in_specs = _edge_specs(2, 1, 0)
    operands = [tr, tc_, ea16]
    if first:
        out_shape.append(jax.ShapeDtypeStruct((E_PAD, 64), _F32))
        out_specs.append(_bspec((BE, 64), _EMAP))
    else:
        in_specs += [_bspec((BE, 64), _EMAP)]
        operands.append(rbf_in)
    in_specs += _wspecs(ws)
    operands += ws

    return pl.pallas_call(
        body, grid=(GRID_E,), in_specs=in_specs, out_specs=out_specs,
        out_shape=out_shape, compiler_params=_ARB,
    )(*operands)


def _tc_node(h, agg_a, agg_b, pd_a, pd_b, pos16, ew, mm128, mm64):
    ws = [ew["wnh"], ew["wna"], ew["wnb"], ew["n0b"],
          ew["nlng"], ew["nlnb"], ew["n1w"], ew["n1b"],
          ew["normg"], ew["normb"], mm128, mm64]

    def body(h_ref, a_ref, b_ref, pda_ref, pdb_ref, p_ref, *rest):
        (wnh, wna, wnb, n0b, nlng, nlnb, n1w, n1b, normg, normb,
         m128, m64) = (r[...] for r in rest[:12])
        hn_ref, p1_ref = rest[12:]
        hv = h_ref[...]
        t = hv @ wnh + a_ref[...] @ wna + b_ref[...] @ wnb + n0b
        t = _lnorm_mx(_silu(t), nlng, nlnb, m128) @ n1w + n1b
        hn_ref[...] = _lnorm_mx(hv + t, normg, normb, m64)
        p1_ref[...] = p_ref[...] + pda_ref[...] + pdb_ref[...]

    return pl.pallas_call(
        body, grid=(GRID_N,),
        in_specs=[_bspec((BN, 64), _EMAP), _bspec((BN, 32), _EMAP),
                  _bspec((BN, 32), _EMAP), _bspec((BN, 16), _EMAP),
                  _bspec((BN, 16), _EMAP), _bspec((BN, 16), _EMAP)]
        + _wspecs(ws),
        out_specs=[_bspec((BN, 64), _EMAP), _bspec((BN, 16), _EMAP)],
        out_shape=[jax.ShapeDtypeStruct((N_PAD, 64), _F32),
                   jax.ShapeDtypeStruct((N_PAD, 16), _F32)],
        compiler_params=_ARB,
    )(h, agg_a, agg_b, pd_a, pd_b, pos16, *ws)


def _mones(d):
    return jnp.full((d, d), 1.0 / d, _F32)


def _tc_fb(pr, pc, ew, sp, first):
    ws = [ew["f0w"], ew["f0b"], ew["f1w"], ew["f1b"], sp["s16"],
          sp["o16_64"]]

    def body(pr_ref, pc_ref, w0, b0, w1, b1, s16, o16_64, *outs):
        diff = pr_ref[...] - pc_ref[...]
        dn16 = jnp.maximum(jnp.sqrt((diff * diff) @ s16[...]), 1e-5)
        fb = _silu((dn16 @ o16_64[...]) * w0[...] + b0[...]) @ w1[...] \
            + b1[...]
        outs[0][0, :, :] = fb[:, 0:32]
        outs[0][1, :, :] = fb[:, 32:64]
        if first:
            outs[1][...] = dn16

    out_shape = [jax.ShapeDtypeStruct((2, E_PAD, 32), _F32)]
    out_specs = [pl.BlockSpec((2, BE, 32), _ACC3)]
    if first:
        out_shape.append(jax.ShapeDtypeStruct((E_PAD, 16), _F32))
        out_specs.append(_bspec((BE, 16), _EMAP))

    res = pl.pallas_call(
        body, grid=(GRID_E,),
        in_specs=_edge_specs(0, 2, 0) + _wspecs(ws),
        out_specs=out_specs, out_shape=out_shape, compiler_params=_ARB,
    )(pr, pc, *ws)
    return res if first else res[0]


def _tc_addf(hn, fa, fb):
    def body(h_ref, a_ref, b_ref, o_ref):
        f = jnp.concatenate([a_ref[...], b_ref[...]], -1)
        o_ref[...] = h_ref[...] + 0.1 * f

    return pl.pallas_call(
        body, grid=(GRID_N,),
        in_specs=[_bspec((BN, 64), _EMAP), _bspec((BN, 32), _EMAP),
                  _bspec((BN, 32), _EMAP)],
        out_specs=_bspec((BN, 64), _EMAP),
        out_shape=jax.ShapeDtypeStruct((N_PAD, 64), _F32),
        compiler_params=_ARB,
    )(hn, fa, fb)


def _tc_cfa(d1, cw, sp):
    ws = [cw["rc"], cw["riw"], cw["rp0w"], cw["rp0b"], cw["rp1w"], cw["rp1b"],
          cw["w0w"], cw["w0b"], cw["w1w"], cw["w1b"], cw["w2w"], cw["w2b"],
          sp["o16_64"]]

    def body(d_ref, *rest):
        (rc, riw, rp0w, rp0b, rp1w, rp1b,
         w0w, w0b, w1w, w1b, w2w, w2b, o16_64) = (r[...] for r in rest[:13])
        o_ref = rest[13]
        rwd = {"c": rc, "inv_w": riw, "p0w": rp0w, "p0b": rp0b,
               "p1w": rp1w, "p1b": rp1b}
        rb = _rbf_tc(d_ref[...] @ o16_64, rwd)
        o_ref[...] = _silu(_silu(rb @ w0w + w0b) @ w1w + w1b) @ w2w + w2b

    return pl.pallas_call(
        body, grid=(GRID_E,),
        in_specs=[_bspec((BE, 16), _EMAP)] + _wspecs(ws),
        out_specs=_bspec((BE, 64), _EMAP),
        out_shape=jax.ShapeDtypeStruct((E_PAD, 64), _F32),
        compiler_params=_ARB,
    )(d1, *ws)


def _tc_cfb(xc, wf, cw):
    ws = [cw["npw"], cw["npb"]]

    def body(x_ref, wf_ref, npw, npb, o_ref):
        msg = (x_ref[...] @ npw[...] + npb[...]) * wf_ref[...]
        o_ref[0, :, :] = msg[:, 0:32]
        o_ref[1, :, :] = msg[:, 32:64]

    return pl.pallas_call(
        body, grid=(GRID_E,),
        in_specs=[_bspec((BE, 64), _EMAP), _bspec((BE, 64), _EMAP)]
        + _wspecs(ws),
        out_specs=pl.BlockSpec((2, BE, 32), _ACC3),
        out_shape=jax.ShapeDtypeStruct((2, E_PAD, 32), _F32),
        compiler_params=_ARB,
    )(xc, wf, *ws)


def _tc_nca(x, ma, mb, batchp, cw, mm64):
    ws = [cw["lng"], cw["lnb"], cw["gw"], cw["gb"], mm64]

    def body(x_ref, a_ref, b_ref, bt_ref, lng, lnb, gw, gb, m64, x1_ref,
             acc_ref):
        s = x_ref[...] + jnp.concatenate([a_ref[...], b_ref[...]], -1)
        out0 = _lnorm_mx(s, lng[...], lnb[...], m64[...])
        x1 = out0 * jax.nn.sigmoid(out0 @ gw[...] + gb[...])
        x1_ref[...] = x1
        oh = (bt_ref[...] == lax.broadcasted_iota(jnp.int32, (BN, 64), 1))
        oh = oh.astype(_F32)
        xa = jnp.concatenate([x1, jnp.ones((BN, 64), _F32)], -1)
        psum = lax.dot_general(oh, xa, (((0,), (0,)), ((), ())),
                               preferred_element_type=_F32)

        @pl.when(pl.program_id(0) == 0)
        def _():
            acc_ref[...] = jnp.zeros_like(acc_ref)

        acc_ref[...] += psum

    return pl.pallas_call(
        body, grid=(GRID_N,),
        in_specs=[_bspec((BN, 64), _EMAP), _bspec((BN, 32), _EMAP),
                  _bspec((BN, 32), _EMAP),
                  pl.BlockSpec((BN, 1), _EMAP)] + _wspecs(ws),
        out_specs=[_bspec((BN, 64), _EMAP), _bspec((64, 128), _WMAP0)],
        out_shape=[jax.ShapeDtypeStruct((N_PAD, 64), _F32),
                   jax.ShapeDtypeStruct((64, 128), _F32)],
        compiler_params=_ARB,
    )(x, ma, mb, batchp, *ws)


def _tc_ncc(x1, batchp, accs, vw):
    ws = [vw["a0w"], vw["a0b"], vw["alng"], vw["alnb"], vw["a1w"], vw["a1b"],
          vw["normg"], vw["normb"], vw["b0w"], vw["b0b"], vw["blng"],
          vw["blnb"]]

    def body(x_ref, bt_ref, acc_ref, *rest):
        (a0w, a0b, alng, alnb, a1w, a1b, normg, normb,
         b0w, b0b, blng, blnb) = (r[...] for r in rest[:12])
        o_ref = rest[12]
        acc = acc_ref[...]
        sums = acc[:, 0:64]
        cnt = acc[:, 64:128]
        mean = sums / jnp.maximum(cnt, 1.0)
        t = _lnorm(_gelu(mean @ a0w + a0b), alng, alnb) @ a1w + a1b
        vnn = _lnorm(t, normg, normb)
        brow = _lnorm(_gelu(vnn @ b0w + b0b), blng, blnb)
        oh = (bt_ref[...] == lax.broadcasted_iota(jnp.int32, (BN, 64), 1))
        o_ref[...] = x_ref[...] + oh.astype(_F32) @ brow

    return pl.pallas_call(
        body, grid=(GRID_N,),
        in_specs=[_bspec((BN, 64), _EMAP), pl.BlockSpec((BN, 1), _EMAP),
                  _bspec((64, 128), _WMAP0)] + _wspecs(ws),
        out_specs=_bspec((BN, 64), _EMAP),
        out_shape=jax.ShapeDtypeStruct((N_PAD, 64), _F32),
        compiler_params=_ARB,
    )(x1, batchp, accs, *ws)


def _tc_final(hn, fa, fb, ow):
    ws = [ow["w"], ow["b"]]

    def body(h_ref, a_ref, b_ref, w_ref, bb_ref, o_ref):
        f = jnp.concatenate([a_ref[...], b_ref[...]], -1)
        h = h_ref[...] + 0.1 * f
        o_ref[...] = h @ w_ref[...] + bb_ref[...]

    return pl.pallas_call(
        body, grid=(GRID_N,),
        in_specs=[_bspec((BN, 64), _EMAP), _bspec((BN, 32), _EMAP),
                  _bspec((BN, 32), _EMAP)] + _wspecs(ws),
        out_specs=_bspec((BN, 64), _EMAP),
        out_shape=jax.ShapeDtypeStruct((N_PAD, 64), _F32),
        compiler_params=_ARB,
    )(hn, fa, fb, *ws)


def _padr(w, rows):
    return jnp.pad(w, ((0, rows - w.shape[0]), (0, 0)))


def _prep_rbf(p):
    return {"c": p["centers"][None, :],
            "inv_w": 1.0 / (jnp.abs(p["widths"]) + 1e-5)[None, :],
            "p0w": p["p0"]["w"], "p0b": p["p0"]["b"][None, :],
            "p1w": p["p1"]["w"], "p1b": p["p1"]["b"][None, :]}


def _prep_egnn(p):
    w = p["e0"]["w"]
    n0 = p["n0"]["w"]
    return {
        "whr": _padr(w[0:64], 80), "whc": _padr(w[64:128], 80),
        "wd": w[128:129], "wea": _padr(w[129:142], 16), "wrb": w[142:206],
        "e0b": p["e0"]["b"][None, :],
        "elng": p["eln"]["g"][None, :], "elnb": p["eln"]["b"][None, :],
        "e1w": p["e1"]["w"], "e1b": p["e1"]["b"][None, :],
        "attw": jnp.tile(p["att"]["w"], (1, 8)),
        "attb": jnp.tile(p["att"]["b"], 8)[None, :],
        "c0w": p["c0"]["w"], "c0b": p["c0"]["b"][None, :],
        "c1w": jnp.tile(p["c1"]["w"], (1, 8)),
        "wnh": n0[0:64], "wna": n0[64:96], "wnb": n0[96:128],
        "n0b": p["n0"]["b"][None, :],
        "nlng": p["nln"]["g"][None, :], "nlnb": p["nln"]["b"][None, :],
        "n1w": p["n1"]["w"], "n1b": p["n1"]["b"][None, :],
        "normg": p["norm"]["g"][None, :], "normb": p["norm"]["b"][None, :],
        "f0w": p["f0"]["w"], "f0b": p["f0"]["b"][None, :],
        "f1w": p["f1"]["w"], "f1b": p["f1"]["b"][None, :],
    }


def _prep_cf(p):
    r = _prep_rbf(p["rbf"])
    return {"rc": r["c"], "riw": r["inv_w"], "rp0w": r["p0w"],
            "rp0b": r["p0b"], "rp1w": r["p1w"], "rp1b": r["p1b"],
            "w0w": p["w0"]["w"], "w0b": p["w0"]["b"][None, :],
            "w1w": p["w1"]["w"], "w1b": p["w1"]["b"][None, :],
            "w2w": p["w2"]["w"], "w2b": p["w2"]["b"][None, :],
            "npw": p["np"]["w"], "npb": p["np"]["b"][None, :],
            "lng": p["ln"]["g"][None, :], "lnb": p["ln"]["b"][None, :],
            "gw": p["gate"]["w"], "gb": p["gate"]["b"][None, :]}


def _prep_vn(p):
    return {"a0w": p["a0"]["w"], "a0b": p["a0"]["b"][None, :],
            "alng": p["aln"]["g"][None, :], "alnb": p["aln"]["b"][None, :],
            "a1w": p["a1"]["w"], "a1b": p["a1"]["b"][None, :],
            "normg": p["norm"]["g"][None, :], "normb": p["norm"]["b"][None, :],
            "b0w": p["b0"]["w"], "b0b": p["b0"]["b"][None, :],
            "blng": p["bln"]["g"][None, :], "blnb": p["bln"]["b"][None, :]}


def kernel(x, pos, edge_index, edge_attr, batch, params):
    row = edge_index[0]
    col = edge_index[1]
    rowp = jnp.pad(row, (0, E_PAD - E), constant_values=N).reshape(-1, EBLK)
    colp = jnp.pad(col, (0, E_PAD - E), constant_values=N).reshape(-1, EBLK)
    ea16 = jnp.pad(edge_attr, ((0, E_PAD - E), (0, 16 - BOND)))
    xp = jnp.pad(x, ((0, N_PAD - N), (0, 48 - IN_DIM)))
    pos16 = jnp.pad(pos, ((0, N_PAD - N), (0, 13)))
    batchp = jnp.pad(batch[:, None], ((0, N_PAD - N), (0, 0)),
                     constant_values=NGRAPH)

    erw = _prep_rbf(params["edge_rbf"])
    iw = {"w": _padr(params["in0"]["w"], 48), "b": params["in0"]["b"][None, :],
          "g": params["inln"]["g"][None, :], "bb": params["inln"]["b"][None, :]}
    b0 = _prep_egnn(params["blocks"][0]["egnn"])
    b1 = _prep_egnn(params["blocks"][1]["egnn"])
    cw = _prep_cf(params["blocks"][0]["cf"])
    vw = _prep_vn(params["blocks"][0]["vn"])
    ow = {"w": params["out"]["w"], "b": params["out"]["b"][None, :]}
    m128 = _mones(128)
    m64 = _mones(64)
    sp = {"s16": jnp.ones((16, 16), _F32),
          "o16_64": jnp.full((16, 64), 1.0 / 16, _F32),
          "o16_128": jnp.full((16, 128), 1.0 / 16, _F32),
          "o8_64": jnp.full((8, 64), 1.0 / 8, _F32),
          "o8_16": jnp.full((8, 16), 1.0 / 8, _F32)}

    h0 = _tc_input(xp, iw, m64)
    t0 = jnp.concatenate([h0, pos16], 1)

    tr, tc_ = _sc_gather(t0, [rowp, colp])
    mm0, pd0, rbfv = _tc_edge(tr, tc_, ea16, None, b0, erw, m128, sp,
                              first=True)
    agg0 = _sc_scatter(mm0, rowp, 32)
    pda0 = _sc_scatter(pd0, rowp, 16, edge_split=True)
    hn0, pos1 = _tc_node(h0, agg0[0], agg0[1], pda0[0], pda0[1], pos16, b0,
                         m128, m64)
    pr, pc = _sc_gather(pos1, [rowp, colp])
    ff0, d1 = _tc_fb(pr, pc, b0, sp, first=True)
    wf = _tc_cfa(d1, cw, sp)
    fagg0 = _sc_scatter(ff0, rowp, 32)
    xcf = _tc_addf(hn0, fagg0[0], fagg0[1])

    (xc,) = _sc_gather(xcf, [colp])
    cm = _tc_cfb(xc, wf, cw)
    cagg = _sc_scatter(cm, rowp, 32)
    x1, accs = _tc_nca(xcf, cagg[0], cagg[1], batchp, cw, m64)
    x2 = _tc_ncc(x1, batchp, accs, vw)

    t2 = jnp.concatenate([x2, pos1], 1)
    tr2, tc2 = _sc_gather(t2, [rowp, colp])
    mm1, pd1 = _tc_edge(tr2, tc2, ea16, rbfv, b1, erw, m128, sp, first=False)
    agg1 = _sc_scatter(mm1, rowp, 32)
    pda1 = _sc_scatter(pd1, rowp, 16, edge_split=True)
    hn1, pos2 = _tc_node(x2, agg1[0], agg1[1], pda1[0], pda1[1], pos1, b1,
                         m128, m64)
    pr2, pc2 = _sc_gather(pos2, [rowp, colp])
    ff1 = _tc_fb(pr2, pc2, b1, sp, first=False)
    fagg1 = _sc_scatter(ff1, rowp, 32)

    out = _tc_final(hn1, fagg1[0], fagg1[1], ow)
    return out[:N]


# speedup vs baseline: 1.0282x; 1.0282x over previous
import math
import functools

import jax
import jax.numpy as jnp
from jax import lax
from jax.experimental import pallas as pl
from jax.experimental.pallas import tpu as pltpu
from jax.experimental.pallas import tpu_sc as plsc

N = 50000
E = 800000
IN_DIM = 47
H = 64
NRBF = 64
BOND = 13
NB = 2
NGRAPH = 64
OUT = 64
CUT = 10.0

NC, NS = 2, 16
NW = NC * NS
EBLK = 128
BLK_PER_W = 196
E_PAD = NW * BLK_PER_W * EBLK
N_PAD = 51200

BE = 1024
GRID_E = E_PAD // BE
BN = 512
GRID_N = N_PAD // BN

_F32 = jnp.float32


K_G = 7


def _sc_gather(tab, idxs):
    D = tab.shape[1]
    n = len(idxs)
    ngrp = BLK_PER_W // K_G
    mesh = plsc.VectorSubcoreMesh(core_axis_name="c", subcore_axis_name="s",
                                  num_cores=NC, num_subcores=NS)
    out_type = [jax.ShapeDtypeStruct((E_PAD, D), _F32)] * n
    scratch = [pltpu.VMEM((BLK_PER_W, EBLK), jnp.int32),
               pltpu.VMEM((K_G, EBLK, D), _F32),
               pltpu.SemaphoreType.DMA,
               pltpu.SemaphoreType.DMA]

    def body(tab_ref, *rest):
        idx_refs = rest[:n]
        out_refs = rest[n:2 * n]
        idxb, rowsb, sem_g, sem_w = rest[2 * n:]
        c = lax.axis_index("c")
        s = lax.axis_index("s")
        w = s * NC + c
        blk0 = w * BLK_PER_W

        for i in range(n):
            pltpu.sync_copy(idx_refs[i].at[pl.ds(blk0, BLK_PER_W), :], idxb)

            def grp(g, carry):
                b0 = g * K_G
                gd = [pltpu.async_copy(tab_ref.at[idxb.at[b0 + k]],
                                       rowsb.at[k], sem_g)
                      for k in range(K_G)]
                for d in gd:
                    d.wait()
                wd = [pltpu.async_copy(
                    rowsb.at[k],
                    out_refs[i].at[pl.ds((blk0 + b0 + k) * EBLK, EBLK), :],
                    sem_w) for k in range(K_G)]
                for d in wd:
                    d.wait()
                return carry

            lax.fori_loop(0, ngrp, grp, 0)

    f = pl.kernel(body, out_type=out_type, mesh=mesh, scratch_types=scratch,
                  compiler_params=pltpu.CompilerParams(
                      use_tc_tiling_on_sc=False))
    return f(tab, *idxs)


def _sc_scatter(msg, idx, dh, edge_split=False):
    p = msg.shape[0]
    rps = N_PAD // NS
    rb = rps // EBLK
    nblk = E_PAD // EBLK
    if edge_split:
        bps = nblk // (NC * NS)
        ks = 7
    else:
        bps = nblk // NS
        ks = 4 if dh >= 32 else 8
    ngrp = bps // ks
    mesh = plsc.VectorSubcoreMesh(core_axis_name="c", subcore_axis_name="s",
                                  num_cores=NC, num_subcores=NS)
    out_type = jax.ShapeDtypeStruct((2, N_PAD, dh), _F32)
    scratch = [pltpu.VMEM_SHARED((N_PAD, dh), _F32),
               pltpu.VMEM((ks, EBLK), jnp.int32),
               pltpu.VMEM((ks, EBLK, dh), _F32),
               pltpu.VMEM((EBLK, dh), _F32),
               pltpu.SemaphoreType.DMA,
               pltpu.SemaphoreType.DMA]
    offs = [o for o in (0, 16, 24, 32, 48) if o + 16 <= dh]

    def body(msg_ref, idx_ref, out_ref, acc, idxb, msgb, zbuf, sem_m, sem_s):
        c = lax.axis_index("c")
        s = lax.axis_index("s")
        plane = c * (p - 1)

        def zrow(j, carry):
            for o in offs:
                zbuf[j, pl.ds(o, 16)] = jnp.zeros((16,), _F32)
            return carry

        lax.fori_loop(0, EBLK, zrow, 0)

        zd = [pltpu.async_copy(zbuf, acc.at[pl.ds(s * rps + r * EBLK, EBLK),
                                            :], sem_m) for r in range(rb)]
        for d in zd:
            d.wait()
        plsc.subcore_barrier()

        if edge_split:
            myblk0 = (c * NS + s) * bps
        else:
            myblk0 = s * bps

        def grp(g, carry):
            b0 = myblk0 + g * ks
            pltpu.sync_copy(idx_ref.at[pl.ds(b0, ks), :], idxb)
            md = [pltpu.async_copy(
                msg_ref.at[plane, pl.ds((b0 + k) * EBLK, EBLK), :],
                msgb.at[k], sem_m) for k in range(ks)]
            for d in md:
                d.wait()
            sd = [pltpu.async_copy(msgb.at[k], acc.at[idxb.at[k]], sem_s,
                                   add=True) for k in range(ks)]
            for d in sd:
                d.wait()
            return carry

        lax.fori_loop(0, ngrp, grp, 0)
        plsc.subcore_barrier()

        cd = [pltpu.async_copy(acc.at[pl.ds(s * rps + r * EBLK, EBLK), :],
                               out_ref.at[c, pl.ds(s * rps + r * EBLK, EBLK),
                                          :], sem_m) for r in range(rb)]
        for d in cd:
            d.wait()

    f = pl.kernel(body, out_type=out_type, mesh=mesh, scratch_types=scratch,
                  compiler_params=pltpu.CompilerParams(
                      use_tc_tiling_on_sc=False))
    return f(msg, idx)


def _silu(t):
    return t * jax.nn.sigmoid(t)


def _gelu(t):
    return 0.5 * t * (1.0 + lax.erf(t * 0.7071067811865476))


def _lnorm(t, g, b):
    m = jnp.mean(t, -1, keepdims=True)
    v = jnp.mean((t - m) ** 2, -1, keepdims=True)
    return (t - m) * lax.rsqrt(v + 1e-5) * g + b


def _lnorm_mx(t, g, b, mones):
    r = t - t @ mones
    v = (r * r) @ mones
    return r * lax.rsqrt(v + 1e-5) * g + b


def _rbf_tc(d, rw):
    env = 0.5 * (jnp.cos(d * (math.pi / CUT)) + 1.0)
    env = env * jnp.where(d < CUT, 1.0, 0.0)
    r = jnp.exp(-0.5 * ((d - rw["c"]) * rw["inv_w"]) ** 2)
    hh = r * env
    return _silu(hh @ rw["p0w"] + rw["p0b"]) @ rw["p1w"] + rw["p1b"]


def _bspec(shape, emap):
    return pl.BlockSpec(shape, emap)


_EMAP = lambda i: (i, 0)
_WMAP0 = lambda i: (0, 0)
_ACC3 = lambda i: (0, i, 0)

_ARB = pltpu.CompilerParams(dimension_semantics=("arbitrary",))


def _edge_specs(n80, n16, n1):
    specs = [_bspec((BE, 80), _EMAP)] * n80
    specs += [_bspec((BE, 16), _EMAP)] * n16
    specs += [_bspec((BE, 1), _EMAP)] * n1
    return specs


def _wspecs(ws):
    return [_bspec(w.shape, _WMAP0) for w in ws]


def _tc_input(xp, iw, mm64):
    ws = [iw["w"], iw["b"], iw["g"], iw["bb"], mm64]

    def body(x_ref, w_ref, b_ref, g_ref, gb_ref, m64_ref, o_ref):
        t = _gelu(x_ref[...] @ w_ref[...] + b_ref[...])
        o_ref[...] = _lnorm_mx(t, g_ref[...], gb_ref[...], m64_ref[...])

    return pl.pallas_call(
        body, grid=(GRID_N,),
        in_specs=[_bspec((BN, 48), _EMAP)] + _wspecs(ws),
        out_specs=_bspec((BN, 64), _EMAP),
        out_shape=jax.ShapeDtypeStruct((N_PAD, 64), _F32),
        compiler_params=_ARB,
    )(xp, *ws)


def _tc_edge(tr, tc_, ea16, rbf_in, ew, rw, mm128, sp, first):
    ws = [ew["whr"], ew["whc"], ew["wd"], ew["wea"], ew["wrb"], ew["e0b"],
          ew["elng"], ew["elnb"], ew["e1w"], ew["e1b"],
          ew["attw"], ew["attb"], ew["c0w"], ew["c0b"], ew["c1w"], mm128,
          sp["s16"], sp["o16_128"], sp["o8_64"], sp["o8_16"]]
    if first:
        ws += [sp["o16_64"], rw["c"], rw["inv_w"], rw["p0w"], rw["p0b"],
               rw["p1w"], rw["p1b"]]
    nw = len(ws)

    def body(tr_ref, tc_ref, ea_ref, *rest):
        if first:
            wrefs = rest[:nw]
            mm_ref, pd_ref, rb_ref = rest[nw:]
        else:
            rb_in_ref = rest[0]
            wrefs = rest[1:1 + nw]
            mm_ref, pd_ref = rest[1 + nw:]
        (whr, whc, wd, wea, wrb, e0b, elng, elnb, e1w, e1b,
         attw, attb, c0w, c0b, c1w, m128, s16, o16_128, o8_64, o8_16) = (
            r[...] for r in wrefs[:20])
        trv = tr_ref[...]
        tcv = tc_ref[...]
        diff16 = trv[:, 64:80] - tcv[:, 64:80]
        dist16 = jnp.maximum(
            jnp.sqrt((diff16 * diff16) @ s16), 1e-5)
        if first:
            o16_64, rc, riw, rp0w, rp0b, rp1w, rp1b = (
                r[...] for r in wrefs[20:])
            rwd = {"c": rc, "inv_w": riw, "p0w": rp0w, "p0b": rp0b,
                   "p1w": rp1w, "p1b": rp1b}
            rb = _rbf_tc(dist16 @ o16_64, rwd)
            rb_ref[...] = rb
        else:
            rb = rb_in_ref[...]
        t = (trv @ whr + tcv @ whc + (dist16 @ o16_128) * wd
             + ea_ref[...] @ wea + rb @ wrb + e0b)
        t = _lnorm_mx(_silu(t), elng, elnb, m128)
        m = _silu(t @ e1w + e1b)
        att = jax.nn.sigmoid(m @ attw + attb)
        matt = m * (att @ o8_64)
        cw = _silu(m @ c0w + c0b) @ c1w
        mm_ref[0, :, :] = matt[:, 0:32]
        mm_ref[1, :, :] = matt[:, 32:64]
        pd_ref[0, :, :] = diff16 * (cw @ o8_16)

    out_shape = [jax.ShapeDtypeStruct((2, E_PAD, 32), _F32),
                 jax.ShapeDtypeStruct((1, E_PAD, 16), _F32)]
    out_specs = [pl.BlockSpec((2, BE, 32), _ACC3),
                 pl.BlockSpec((1, BE, 16), _ACC3)]
    in_specs = _edge_specs(2, 1, 0)
    operands = [tr, tc_, ea16]
    if first:
        out_shape.append(jax.ShapeDtypeStruct((E_PAD, 64), _F32))
        out_specs.append(_bspec((BE, 64), _EMAP))
    else:
        in_specs += [_bspec((BE, 64), _EMAP)]
        operands.append(rbf_in)
    in_specs += _wspecs(ws)
    operands += ws

    return pl.pallas_call(
        body, grid=(GRID_E,), in_specs=in_specs, out_specs=out_specs,
        out_shape=out_shape, compiler_params=_ARB,
    )(*operands)


def _tc_node(h, agg_a, agg_b, pd_a, pd_b, pos16, ew, mm128, mm64):
    ws = [ew["wnh"], ew["wna"], ew["wnb"], ew["n0b"],
          ew["nlng"], ew["nlnb"], ew["n1w"], ew["n1b"],
          ew["normg"], ew["normb"], mm128, mm64]

    def body(h_ref, a_ref, b_ref, pda_ref, pdb_ref, p_ref, *rest):
        (wnh, wna, wnb, n0b, nlng, nlnb, n1w, n1b, normg, normb,
         m128, m64) = (r[...] for r in rest[:12])
        hn_ref, p1_ref = rest[12:]
        hv = h_ref[...]
        t = hv @ wnh + a_ref[...] @ wna + b_ref[...] @ wnb + n0b
        t = _lnorm_mx(_silu(t), nlng, nlnb, m128) @ n1w + n1b
        hn_ref[...] = _lnorm_mx(hv + t, normg, normb, m64)
        p1_ref[...] = p_ref[...] + pda_ref[...] + pdb_ref[...]

    return pl.pallas_call(
        body, grid=(GRID_N,),
        in_specs=[_bspec((BN, 64), _EMAP), _bspec((BN, 32), _EMAP),
                  _bspec((BN, 32), _EMAP), _bspec((BN, 16), _EMAP),
                  _bspec((BN, 16), _EMAP), _bspec((BN, 16), _EMAP)]
        + _wspecs(ws),
        out_specs=[_bspec((BN, 64), _EMAP), _bspec((BN, 16), _EMAP)],
        out_shape=[jax.ShapeDtypeStruct((N_PAD, 64), _F32),
                   jax.ShapeDtypeStruct((N_PAD, 16), _F32)],
        compiler_params=_ARB,
    )(h, agg_a, agg_b, pd_a, pd_b, pos16, *ws)


def _mones(d):
    return jnp.full((d, d), 1.0 / d, _F32)


def _tc_fb(pr, pc, ew, sp, first):
    ws = [ew["f0w"], ew["f0b"], ew["f1w"], ew["f1b"], sp["s16"],
          sp["o16_64"]]

    def body(pr_ref, pc_ref, w0, b0, w1, b1, s16, o16_64, *outs):
        diff = pr_ref[...] - pc_ref[...]
        dn16 = jnp.maximum(jnp.sqrt((diff * diff) @ s16[...]), 1e-5)
        fb = _silu((dn16 @ o16_64[...]) * w0[...] + b0[...]) @ w1[...] \
            + b1[...]
        outs[0][0, :, :] = fb[:, 0:32]
        outs[0][1, :, :] = fb[:, 32:64]
        if first:
            outs[1][...] = dn16

    out_shape = [jax.ShapeDtypeStruct((2, E_PAD, 32), _F32)]
    out_specs = [pl.BlockSpec((2, BE, 32), _ACC3)]
    if first:
        out_shape.append(jax.ShapeDtypeStruct((E_PAD, 16), _F32))
        out_specs.append(_bspec((BE, 16), _EMAP))

    res = pl.pallas_call(
        body, grid=(GRID_E,),
        in_specs=_edge_specs(0, 2, 0) + _wspecs(ws),
        out_specs=out_specs, out_shape=out_shape, compiler_params=_ARB,
    )(pr, pc, *ws)
    return res if first else res[0]


def _tc_addf(hn, fa, fb):
    def body(h_ref, a_ref, b_ref, o_ref):
        f = jnp.concatenate([a_ref[...], b_ref[...]], -1)
        o_ref[...] = h_ref[...] + 0.1 * f

    return pl.pallas_call(
        body, grid=(GRID_N,),
        in_specs=[_bspec((BN, 64), _EMAP), _bspec((BN, 32), _EMAP),
                  _bspec((BN, 32), _EMAP)],
        out_specs=_bspec((BN, 64), _EMAP),
        out_shape=jax.ShapeDtypeStruct((N_PAD, 64), _F32),
        compiler_params=_ARB,
    )(hn, fa, fb)


def _tc_cf(xc, d1, cw, sp):
    ws = [cw["rc"], cw["riw"], cw["rp0w"], cw["rp0b"], cw["rp1w"], cw["rp1b"],
          cw["w0w"], cw["w0b"], cw["w1w"], cw["w1b"], cw["w2w"], cw["w2b"],
          cw["npw"], cw["npb"], sp["o16_64"]]

    def body(x_ref, d_ref, *rest):
        (rc, riw, rp0w, rp0b, rp1w, rp1b,
         w0w, w0b, w1w, w1b, w2w, w2b, npw, npb, o16_64) = (
            r[...] for r in rest[:15])
        o_ref = rest[15]
        rwd = {"c": rc, "inv_w": riw, "p0w": rp0w, "p0b": rp0b,
               "p1w": rp1w, "p1b": rp1b}
        rb = _rbf_tc(d_ref[...] @ o16_64, rwd)
        wf = _silu(_silu(rb @ w0w + w0b) @ w1w + w1b) @ w2w + w2b
        msg = (x_ref[...] @ npw + npb) * wf
        o_ref[0, :, :] = msg[:, 0:32]
        o_ref[1, :, :] = msg[:, 32:64]

    return pl.pallas_call(
        body, grid=(GRID_E,),
        in_specs=[_bspec((BE, 64), _EMAP), _bspec((BE, 16), _EMAP)]
        + _wspecs(ws),
        out_specs=pl.BlockSpec((2, BE, 32), _ACC3),
        out_shape=jax.ShapeDtypeStruct((2, E_PAD, 32), _F32),
        compiler_params=_ARB,
    )(xc, d1, *ws)


def _tc_nca(x, ma, mb, batchp, cw, mm64):
    ws = [cw["lng"], cw["lnb"], cw["gw"], cw["gb"], mm64]

    def body(x_ref, a_ref, b_ref, bt_ref, lng, lnb, gw, gb, m64, x1_ref,
             acc_ref):
        s = x_ref[...] + jnp.concatenate([a_ref[...], b_ref[...]], -1)
        out0 = _lnorm_mx(s, lng[...], lnb[...], m64[...])
        x1 = out0 * jax.nn.sigmoid(out0 @ gw[...] + gb[...])
        x1_ref[...] = x1
        oh = (bt_ref[...] == lax.broadcasted_iota(jnp.int32, (BN, 64), 1))
        oh = oh.astype(_F32)
        xa = jnp.concatenate([x1, jnp.ones((BN, 64), _F32)], -1)
        psum = lax.dot_general(oh, xa, (((0,), (0,)), ((), ())),
                               preferred_element_type=_F32)

        @pl.when(pl.program_id(0) == 0)
        def _():
            acc_ref[...] = jnp.zeros_like(acc_ref)

        acc_ref[...] += psum

    return pl.pallas_call(
        body, grid=(GRID_N,),
        in_specs=[_bspec((BN, 64), _EMAP), _bspec((BN, 32), _EMAP),
                  _bspec((BN, 32), _EMAP),
                  pl.BlockSpec((BN, 1), _EMAP)] + _wspecs(ws),
        out_specs=[_bspec((BN, 64), _EMAP), _bspec((64, 128), _WMAP0)],
        out_shape=[jax.ShapeDtypeStruct((N_PAD, 64), _F32),
                   jax.ShapeDtypeStruct((64, 128), _F32)],
        compiler_params=_ARB,
    )(x, ma, mb, batchp, *ws)


def _tc_ncc(x1, batchp, accs, vw):
    ws = [vw["a0w"], vw["a0b"], vw["alng"], vw["alnb"], vw["a1w"], vw["a1b"],
          vw["normg"], vw["normb"], vw["b0w"], vw["b0b"], vw["blng"],
          vw["blnb"]]

    def body(x_ref, bt_ref, acc_ref, *rest):
        (a0w, a0b, alng, alnb, a1w, a1b, normg, normb,
         b0w, b0b, blng, blnb) = (r[...] for r in rest[:12])
        o_ref = rest[12]
        acc = acc_ref[...]
        sums = acc[:, 0:64]
        cnt = acc[:, 64:128]
        mean = sums / jnp.maximum(cnt, 1.0)
        t = _lnorm(_gelu(mean @ a0w + a0b), alng, alnb) @ a1w + a1b
        vnn = _lnorm(t, normg, normb)
        brow = _lnorm(_gelu(vnn @ b0w + b0b), blng, blnb)
        oh = (bt_ref[...] == lax.broadcasted_iota(jnp.int32, (BN, 64), 1))
        o_ref[...] = x_ref[...] + oh.astype(_F32) @ brow

    return pl.pallas_call(
        body, grid=(GRID_N,),
        in_specs=[_bspec((BN, 64), _EMAP), pl.BlockSpec((BN, 1), _EMAP),
                  _bspec((64, 128), _WMAP0)] + _wspecs(ws),
        out_specs=_bspec((BN, 64), _EMAP),
        out_shape=jax.ShapeDtypeStruct((N_PAD, 64), _F32),
        compiler_params=_ARB,
    )(x1, batchp, accs, *ws)


def _tc_final(hn, fa, fb, ow):
    ws = [ow["w"], ow["b"]]

    def body(h_ref, a_ref, b_ref, w_ref, bb_ref, o_ref):
        f = jnp.concatenate([a_ref[...], b_ref[...]], -1)
        h = h_ref[...] + 0.1 * f
        o_ref[...] = h @ w_ref[...] + bb_ref[...]

    return pl.pallas_call(
        body, grid=(GRID_N,),
        in_specs=[_bspec((BN, 64), _EMAP), _bspec((BN, 32), _EMAP),
                  _bspec((BN, 32), _EMAP)] + _wspecs(ws),
        out_specs=_bspec((BN, 64), _EMAP),
        out_shape=jax.ShapeDtypeStruct((N_PAD, 64), _F32),
        compiler_params=_ARB,
    )(hn, fa, fb, *ws)


def _padr(w, rows):
    return jnp.pad(w, ((0, rows - w.shape[0]), (0, 0)))


def _prep_rbf(p):
    return {"c": p["centers"][None, :],
            "inv_w": 1.0 / (jnp.abs(p["widths"]) + 1e-5)[None, :],
            "p0w": p["p0"]["w"], "p0b": p["p0"]["b"][None, :],
            "p1w": p["p1"]["w"], "p1b": p["p1"]["b"][None, :]}


def _prep_egnn(p):
    w = p["e0"]["w"]
    n0 = p["n0"]["w"]
    return {
        "whr": _padr(w[0:64], 80), "whc": _padr(w[64:128], 80),
        "wd": w[128:129], "wea": _padr(w[129:142], 16), "wrb": w[142:206],
        "e0b": p["e0"]["b"][None, :],
        "elng": p["eln"]["g"][None, :], "elnb": p["eln"]["b"][None, :],
        "e1w": p["e1"]["w"], "e1b": p["e1"]["b"][None, :],
        "attw": jnp.tile(p["att"]["w"], (1, 8)),
        "attb": jnp.tile(p["att"]["b"], 8)[None, :],
        "c0w": p["c0"]["w"], "c0b": p["c0"]["b"][None, :],
        "c1w": jnp.tile(p["c1"]["w"], (1, 8)),
        "wnh": n0[0:64], "wna": n0[64:96], "wnb": n0[96:128],
        "n0b": p["n0"]["b"][None, :],
        "nlng": p["nln"]["g"][None, :], "nlnb": p["nln"]["b"][None, :],
        "n1w": p["n1"]["w"], "n1b": p["n1"]["b"][None, :],
        "normg": p["norm"]["g"][None, :], "normb": p["norm"]["b"][None, :],
        "f0w": p["f0"]["w"], "f0b": p["f0"]["b"][None, :],
        "f1w": p["f1"]["w"], "f1b": p["f1"]["b"][None, :],
    }


def _prep_cf(p):
    r = _prep_rbf(p["rbf"])
    return {"rc": r["c"], "riw": r["inv_w"], "rp0w": r["p0w"],
            "rp0b": r["p0b"], "rp1w": r["p1w"], "rp1b": r["p1b"],
            "w0w": p["w0"]["w"], "w0b": p["w0"]["b"][None, :],
            "w1w": p["w1"]["w"], "w1b": p["w1"]["b"][None, :],
            "w2w": p["w2"]["w"], "w2b": p["w2"]["b"][None, :],
            "npw": p["np"]["w"], "npb": p["np"]["b"][None, :],
            "lng": p["ln"]["g"][None, :], "lnb": p["ln"]["b"][None, :],
            "gw": p["gate"]["w"], "gb": p["gate"]["b"][None, :]}


def _prep_vn(p):
    return {"a0w": p["a0"]["w"], "a0b": p["a0"]["b"][None, :],
            "alng": p["aln"]["g"][None, :], "alnb": p["aln"]["b"][None, :],
            "a1w": p["a1"]["w"], "a1b": p["a1"]["b"][None, :],
            "normg": p["norm"]["g"][None, :], "normb": p["norm"]["b"][None, :],
            "b0w": p["b0"]["w"], "b0b": p["b0"]["b"][None, :],
            "blng": p["bln"]["g"][None, :], "blnb": p["bln"]["b"][None, :]}


def kernel(x, pos, edge_index, edge_attr, batch, params):
    row = edge_index[0]
    col = edge_index[1]
    rowp = jnp.pad(row, (0, E_PAD - E), constant_values=N).reshape(-1, EBLK)
    colp = jnp.pad(col, (0, E_PAD - E), constant_values=N).reshape(-1, EBLK)
    ea16 = jnp.pad(edge_attr, ((0, E_PAD - E), (0, 16 - BOND)))
    xp = jnp.pad(x, ((0, N_PAD - N), (0, 48 - IN_DIM)))
    pos16 = jnp.pad(pos, ((0, N_PAD - N), (0, 13)))
    batchp = jnp.pad(batch[:, None], ((0, N_PAD - N), (0, 0)),
                     constant_values=NGRAPH)

    erw = _prep_rbf(params["edge_rbf"])
    iw = {"w": _padr(params["in0"]["w"], 48), "b": params["in0"]["b"][None, :],
          "g": params["inln"]["g"][None, :], "bb": params["inln"]["b"][None, :]}
    b0 = _prep_egnn(params["blocks"][0]["egnn"])
    b1 = _prep_egnn(params["blocks"][1]["egnn"])
    cw = _prep_cf(params["blocks"][0]["cf"])
    vw = _prep_vn(params["blocks"][0]["vn"])
    ow = {"w": params["out"]["w"], "b": params["out"]["b"][None, :]}
    m128 = _mones(128)
    m64 = _mones(64)
    sp = {"s16": jnp.ones((16, 16), _F32),
          "o16_64": jnp.full((16, 64), 1.0 / 16, _F32),
          "o16_128": jnp.full((16, 128), 1.0 / 16, _F32),
          "o8_64": jnp.full((8, 64), 1.0 / 8, _F32),
          "o8_16": jnp.full((8, 16), 1.0 / 8, _F32)}

    h0 = _tc_input(xp, iw, m64)
    t0 = jnp.concatenate([h0, pos16], 1)

    tr, tc_ = _sc_gather(t0, [rowp, colp])
    mm0, pd0, rbfv = _tc_edge(tr, tc_, ea16, None, b0, erw, m128, sp,
                              first=True)
    agg0 = _sc_scatter(mm0, rowp, 32)
    pda0 = _sc_scatter(pd0, rowp, 16, edge_split=True)
    hn0, pos1 = _tc_node(h0, agg0[0], agg0[1], pda0[0], pda0[1], pos16, b0,
                         m128, m64)
    pr, pc = _sc_gather(pos1, [rowp, colp])
    ff0, d1 = _tc_fb(pr, pc, b0, sp, first=True)
    fagg0 = _sc_scatter(ff0, rowp, 32)
    xcf = _tc_addf(hn0, fagg0[0], fagg0[1])

    (xc,) = _sc_gather(xcf, [colp])
    cm = _tc_cf(xc, d1, cw, sp)
    cagg = _sc_scatter(cm, rowp, 32)
    x1, accs = _tc_nca(xcf, cagg[0], cagg[1], batchp, cw, m64)
    x2 = _tc_ncc(x1, batchp, accs, vw)

    t2 = jnp.concatenate([x2, pos1], 1)
    tr2, tc2 = _sc_gather(t2, [rowp, colp])
    mm1, pd1 = _tc_edge(tr2, tc2, ea16, rbfv, b1, erw, m128, sp, first=False)
    agg1 = _sc_scatter(mm1, rowp, 32)
    pda1 = _sc_scatter(pd1, rowp, 16, edge_split=True)
    hn1, pos2 = _tc_node(x2, agg1[0], agg1[1], pda1[0], pda1[1], pos1, b1,
                         m128, m64)
    pr2, pc2 = _sc_gather(pos2, [rowp, colp])
    ff1 = _tc_fb(pr2, pc2, b1, sp, first=False)
    fagg1 = _sc_scatter(ff1, rowp, 32)

    out = _tc_final(hn1, fagg1[0], fagg1[1], ow)
    return out[:N]


# speedup vs baseline: 1.1038x; 1.0736x over previous
import math
import functools

import jax
import jax.numpy as jnp
from jax import lax
from jax.experimental import pallas as pl
from jax.experimental.pallas import tpu as pltpu
from jax.experimental.pallas import tpu_sc as plsc

N = 50000
E = 800000
IN_DIM = 47
H = 64
NRBF = 64
BOND = 13
NB = 2
NGRAPH = 64
OUT = 64
CUT = 10.0

NC, NS = 2, 16
NW = NC * NS
EBLK = 128
BLK_PER_W = 196
E_PAD = NW * BLK_PER_W * EBLK
N_PAD = 51200

BE = 2048
GRID_E = E_PAD // BE
BN = 512
GRID_N = N_PAD // BN

_F32 = jnp.float32


K_G = 7


def _sc_gather(tab, idxs):
    D = tab.shape[1]
    n = len(idxs)
    ngrp = BLK_PER_W // K_G
    mesh = plsc.VectorSubcoreMesh(core_axis_name="c", subcore_axis_name="s",
                                  num_cores=NC, num_subcores=NS)
    out_type = [jax.ShapeDtypeStruct((E_PAD, D), _F32)] * n
    scratch = [pltpu.VMEM((BLK_PER_W, EBLK), jnp.int32),
               pltpu.VMEM((K_G, EBLK, D), _F32),
               pltpu.SemaphoreType.DMA,
               pltpu.SemaphoreType.DMA]

    def body(tab_ref, *rest):
        idx_refs = rest[:n]
        out_refs = rest[n:2 * n]
        idxb, rowsb, sem_g, sem_w = rest[2 * n:]
        c = lax.axis_index("c")
        s = lax.axis_index("s")
        w = s * NC + c
        blk0 = w * BLK_PER_W

        for i in range(n):
            pltpu.sync_copy(idx_refs[i].at[pl.ds(blk0, BLK_PER_W), :], idxb)

            def grp(g, carry):
                b0 = g * K_G
                gd = [pltpu.async_copy(tab_ref.at[idxb.at[b0 + k]],
                                       rowsb.at[k], sem_g)
                      for k in range(K_G)]
                for d in gd:
                    d.wait()
                wd = [pltpu.async_copy(
                    rowsb.at[k],
                    out_refs[i].at[pl.ds((blk0 + b0 + k) * EBLK, EBLK), :],
                    sem_w) for k in range(K_G)]
                for d in wd:
                    d.wait()
                return carry

            lax.fori_loop(0, ngrp, grp, 0)

    f = pl.kernel(body, out_type=out_type, mesh=mesh, scratch_types=scratch,
                  compiler_params=pltpu.CompilerParams(
                      use_tc_tiling_on_sc=False))
    return f(tab, *idxs)


def _sc_scatter(msg, idx, dh, edge_split=False):
    p = msg.shape[0]
    rps = N_PAD // NS
    rb = rps // EBLK
    nblk = E_PAD // EBLK
    if edge_split:
        bps = nblk // (NC * NS)
        ks = 7
    else:
        bps = nblk // NS
        ks = 4 if dh >= 32 else 8
    ngrp = bps // ks
    mesh = plsc.VectorSubcoreMesh(core_axis_name="c", subcore_axis_name="s",
                                  num_cores=NC, num_subcores=NS)
    out_type = jax.ShapeDtypeStruct((2, N_PAD, dh), _F32)
    scratch = [pltpu.VMEM_SHARED((N_PAD, dh), _F32),
               pltpu.VMEM((ks, EBLK), jnp.int32),
               pltpu.VMEM((ks, EBLK, dh), _F32),
               pltpu.VMEM((EBLK, dh), _F32),
               pltpu.SemaphoreType.DMA,
               pltpu.SemaphoreType.DMA]
    offs = [o for o in (0, 16, 24, 32, 48) if o + 16 <= dh]

    def body(msg_ref, idx_ref, out_ref, acc, idxb, msgb, zbuf, sem_m, sem_s):
        c = lax.axis_index("c")
        s = lax.axis_index("s")
        plane = c * (p - 1)

        def zrow(j, carry):
            for o in offs:
                zbuf[j, pl.ds(o, 16)] = jnp.zeros((16,), _F32)
            return carry

        lax.fori_loop(0, EBLK, zrow, 0)

        zd = [pltpu.async_copy(zbuf, acc.at[pl.ds(s * rps + r * EBLK, EBLK),
                                            :], sem_m) for r in range(rb)]
        for d in zd:
            d.wait()
        plsc.subcore_barrier()

        if edge_split:
            myblk0 = (c * NS + s) * bps
        else:
            myblk0 = s * bps

        def grp(g, carry):
            b0 = myblk0 + g * ks
            pltpu.sync_copy(idx_ref.at[pl.ds(b0, ks), :], idxb)
            md = [pltpu.async_copy(
                msg_ref.at[plane, pl.ds((b0 + k) * EBLK, EBLK), :],
                msgb.at[k], sem_m) for k in range(ks)]
            for d in md:
                d.wait()
            sd = [pltpu.async_copy(msgb.at[k], acc.at[idxb.at[k]], sem_s,
                                   add=True) for k in range(ks)]
            for d in sd:
                d.wait()
            return carry

        lax.fori_loop(0, ngrp, grp, 0)
        plsc.subcore_barrier()

        cd = [pltpu.async_copy(acc.at[pl.ds(s * rps + r * EBLK, EBLK), :],
                               out_ref.at[c, pl.ds(s * rps + r * EBLK, EBLK),
                                          :], sem_m) for r in range(rb)]
        for d in cd:
            d.wait()

    f = pl.kernel(body, out_type=out_type, mesh=mesh, scratch_types=scratch,
                  compiler_params=pltpu.CompilerParams(
                      use_tc_tiling_on_sc=False))
    return f(msg, idx)


def _silu(t):
    return t * jax.nn.sigmoid(t)


def _gelu(t):
    return 0.5 * t * (1.0 + lax.erf(t * 0.7071067811865476))


def _lnorm(t, g, b):
    m = jnp.mean(t, -1, keepdims=True)
    v = jnp.mean((t - m) ** 2, -1, keepdims=True)
    return (t - m) * lax.rsqrt(v + 1e-5) * g + b


def _lnorm_mx(t, g, b, mones):
    r = t - t @ mones
    v = (r * r) @ mones
    return r * lax.rsqrt(v + 1e-5) * g + b


def _rbf_tc(d, rw):
    env = 0.5 * (jnp.cos(d * (math.pi / CUT)) + 1.0)
    env = env * jnp.where(d < CUT, 1.0, 0.0)
    r = jnp.exp(-0.5 * ((d - rw["c"]) * rw["inv_w"]) ** 2)
    hh = r * env
    return _silu(hh @ rw["p0w"] + rw["p0b"]) @ rw["p1w"] + rw["p1b"]


def _bspec(shape, emap):
    return pl.BlockSpec(shape, emap)


_EMAP = lambda i: (i, 0)
_WMAP0 = lambda i: (0, 0)
_ACC3 = lambda i: (0, i, 0)

_ARB = pltpu.CompilerParams(dimension_semantics=("arbitrary",))


def _edge_specs(n80, n16, n1):
    specs = [_bspec((BE, 80), _EMAP)] * n80
    specs += [_bspec((BE, 16), _EMAP)] * n16
    specs += [_bspec((BE, 1), _EMAP)] * n1
    return specs


def _wspecs(ws):
    return [_bspec(w.shape, _WMAP0) for w in ws]


def _tc_input(xp, iw, mm64):
    ws = [iw["w"], iw["b"], iw["g"], iw["bb"], mm64]

    def body(x_ref, w_ref, b_ref, g_ref, gb_ref, m64_ref, o_ref):
        t = _gelu(x_ref[...] @ w_ref[...] + b_ref[...])
        o_ref[...] = _lnorm_mx(t, g_ref[...], gb_ref[...], m64_ref[...])

    return pl.pallas_call(
        body, grid=(GRID_N,),
        in_specs=[_bspec((BN, 48), _EMAP)] + _wspecs(ws),
        out_specs=_bspec((BN, 64), _EMAP),
        out_shape=jax.ShapeDtypeStruct((N_PAD, 64), _F32),
        compiler_params=_ARB,
    )(xp, *ws)


def _tc_edge(tr, tc_, ea16, rbf_in, ew, rw, mm128, sp, first):
    ws = [ew["whr"], ew["whc"], ew["wd"], ew["wea"], ew["wrb"], ew["e0b"],
          ew["elng"], ew["elnb"], ew["e1w"], ew["e1b"],
          ew["attw"], ew["attb"], ew["c0w"], ew["c0b"], ew["c1w"], mm128,
          sp["s16"], sp["o16_128"], sp["o8_64"], sp["o8_16"]]
    if first:
        ws += [sp["o16_64"], rw["c"], rw["inv_w"], rw["p0w"], rw["p0b"],
               rw["p1w"], rw["p1b"]]
    nw = len(ws)

    def body(tr_ref, tc_ref, ea_ref, *rest):
        if first:
            wrefs = rest[:nw]
            mm_ref, pd_ref, rb_ref = rest[nw:]
        else:
            rb_in_ref = rest[0]
            wrefs = rest[1:1 + nw]
            mm_ref, pd_ref = rest[1 + nw:]
        (whr, whc, wd, wea, wrb, e0b, elng, elnb, e1w, e1b,
         attw, attb, c0w, c0b, c1w, m128, s16, o16_128, o8_64, o8_16) = (
            r[...] for r in wrefs[:20])
        trv = tr_ref[...]
        tcv = tc_ref[...]
        diff16 = trv[:, 64:80] - tcv[:, 64:80]
        dist16 = jnp.maximum(
            jnp.sqrt((diff16 * diff16) @ s16), 1e-5)
        if first:
            o16_64, rc, riw, rp0w, rp0b, rp1w, rp1b = (
                r[...] for r in wrefs[20:])
            rwd = {"c": rc, "inv_w": riw, "p0w": rp0w, "p0b": rp0b,
                   "p1w": rp1w, "p1b": rp1b}
            rb = _rbf_tc(dist16 @ o16_64, rwd)
            rb_ref[...] = rb
        else:
            rb = rb_in_ref[...]
        t = (trv @ whr + tcv @ whc + (dist16 @ o16_128) * wd
             + ea_ref[...] @ wea + rb @ wrb + e0b)
        t = _lnorm_mx(_silu(t), elng, elnb, m128)
        m = _silu(t @ e1w + e1b)
        att = jax.nn.sigmoid(m @ attw + attb)
        matt = m * (att @ o8_64)
        cw = _silu(m @ c0w + c0b) @ c1w
        mm_ref[0, :, :] = matt[:, 0:32]
        mm_ref[1, :, :] = matt[:, 32:64]
        pd_ref[0, :, :] = diff16 * (cw @ o8_16)

    out_shape = [jax.ShapeDtypeStruct((2, E_PAD, 32), _F32),
                 jax.ShapeDtypeStruct((1, E_PAD, 16), _F32)]
    out_specs = [pl.BlockSpec((2, BE, 32), _ACC3),
                 pl.BlockSpec((1, BE, 16), _ACC3)]
    in_specs = _edge_specs(2, 1, 0)
    operands = [tr, tc_, ea16]
    if first:
        out_shape.append(jax.ShapeDtypeStruct((E_PAD, 64), _F32))
        out_specs.append(_bspec((BE, 64), _EMAP))
    else:
        in_specs += [_bspec((BE, 64), _EMAP)]
        operands.append(rbf_in)
    in_specs += _wspecs(ws)
    operands += ws

    return pl.pallas_call(
        body, grid=(GRID_E,), in_specs=in_specs, out_specs=out_specs,
        out_shape=out_shape, compiler_params=_ARB,
    )(*operands)


def _tc_node(h, agg_a, agg_b, pd_a, pd_b, pos16, ew, mm128, mm64):
    ws = [ew["wnh"], ew["wna"], ew["wnb"], ew["n0b"],
          ew["nlng"], ew["nlnb"], ew["n1w"], ew["n1b"],
          ew["normg"], ew["normb"], mm128, mm64]

    def body(h_ref, a_ref, b_ref, pda_ref, pdb_ref, p_ref, *rest):
        (wnh, wna, wnb, n0b, nlng, nlnb, n1w, n1b, normg, normb,
         m128, m64) = (r[...] for r in rest[:12])
        hn_ref, p1_ref = rest[12:]
        hv = h_ref[...]
        t = hv @ wnh + a_ref[...] @ wna + b_ref[...] @ wnb + n0b
        t = _lnorm_mx(_silu(t), nlng, nlnb, m128) @ n1w + n1b
        hn_ref[...] = _lnorm_mx(hv + t, normg, normb, m64)
        p1_ref[...] = p_ref[...] + pda_ref[...] + pdb_ref[...]

    return pl.pallas_call(
        body, grid=(GRID_N,),
        in_specs=[_bspec((BN, 64), _EMAP), _bspec((BN, 32), _EMAP),
                  _bspec((BN, 32), _EMAP), _bspec((BN, 16), _EMAP),
                  _bspec((BN, 16), _EMAP), _bspec((BN, 16), _EMAP)]
        + _wspecs(ws),
        out_specs=[_bspec((BN, 64), _EMAP), _bspec((BN, 16), _EMAP)],
        out_shape=[jax.ShapeDtypeStruct((N_PAD, 64), _F32),
                   jax.ShapeDtypeStruct((N_PAD, 16), _F32)],
        compiler_params=_ARB,
    )(h, agg_a, agg_b, pd_a, pd_b, pos16, *ws)


def _mones(d):
    return jnp.full((d, d), 1.0 / d, _F32)


def _tc_fb(pr, pc, ew, sp, first):
    ws = [ew["f0w"], ew["f0b"], ew["f1w"], ew["f1b"], sp["s16"],
          sp["o16_64"]]

    def body(pr_ref, pc_ref, w0, b0, w1, b1, s16, o16_64, *outs):
        diff = pr_ref[...] - pc_ref[...]
        dn16 = jnp.maximum(jnp.sqrt((diff * diff) @ s16[...]), 1e-5)
        fb = _silu((dn16 @ o16_64[...]) * w0[...] + b0[...]) @ w1[...] \
            + b1[...]
        outs[0][0, :, :] = fb[:, 0:32]
        outs[0][1, :, :] = fb[:, 32:64]
        if first:
            outs[1][...] = dn16

    out_shape = [jax.ShapeDtypeStruct((2, E_PAD, 32), _F32)]
    out_specs = [pl.BlockSpec((2, BE, 32), _ACC3)]
    if first:
        out_shape.append(jax.ShapeDtypeStruct((E_PAD, 16), _F32))
        out_specs.append(_bspec((BE, 16), _EMAP))

    res = pl.pallas_call(
        body, grid=(GRID_E,),
        in_specs=_edge_specs(0, 2, 0) + _wspecs(ws),
        out_specs=out_specs, out_shape=out_shape, compiler_params=_ARB,
    )(pr, pc, *ws)
    return res if first else res[0]


def _tc_addf(hn, fa, fb):
    def body(h_ref, a_ref, b_ref, o_ref):
        f = jnp.concatenate([a_ref[...], b_ref[...]], -1)
        o_ref[...] = h_ref[...] + 0.1 * f

    return pl.pallas_call(
        body, grid=(GRID_N,),
        in_specs=[_bspec((BN, 64), _EMAP), _bspec((BN, 32), _EMAP),
                  _bspec((BN, 32), _EMAP)],
        out_specs=_bspec((BN, 64), _EMAP),
        out_shape=jax.ShapeDtypeStruct((N_PAD, 64), _F32),
        compiler_params=_ARB,
    )(hn, fa, fb)


def _tc_cf(xc, d1, cw, sp):
    ws = [cw["rc"], cw["riw"], cw["rp0w"], cw["rp0b"], cw["rp1w"], cw["rp1b"],
          cw["w0w"], cw["w0b"], cw["w1w"], cw["w1b"], cw["w2w"], cw["w2b"],
          cw["npw"], cw["npb"], sp["o16_64"]]

    def body(x_ref, d_ref, *rest):
        (rc, riw, rp0w, rp0b, rp1w, rp1b,
         w0w, w0b, w1w, w1b, w2w, w2b, npw, npb, o16_64) = (
            r[...] for r in rest[:15])
        o_ref = rest[15]
        rwd = {"c": rc, "inv_w": riw, "p0w": rp0w, "p0b": rp0b,
               "p1w": rp1w, "p1b": rp1b}
        rb = _rbf_tc(d_ref[...] @ o16_64, rwd)
        wf = _silu(_silu(rb @ w0w + w0b) @ w1w + w1b) @ w2w + w2b
        msg = (x_ref[...] @ npw + npb) * wf
        o_ref[0, :, :] = msg[:, 0:32]
        o_ref[1, :, :] = msg[:, 32:64]

    return pl.pallas_call(
        body, grid=(GRID_E,),
        in_specs=[_bspec((BE, 64), _EMAP), _bspec((BE, 16), _EMAP)]
        + _wspecs(ws),
        out_specs=pl.BlockSpec((2, BE, 32), _ACC3),
        out_shape=jax.ShapeDtypeStruct((2, E_PAD, 32), _F32),
        compiler_params=_ARB,
    )(xc, d1, *ws)


def _tc_nca(x, ma, mb, batchp, cw, mm64):
    ws = [cw["lng"], cw["lnb"], cw["gw"], cw["gb"], mm64]

    def body(x_ref, a_ref, b_ref, bt_ref, lng, lnb, gw, gb, m64, x1_ref,
             acc_ref):
        s = x_ref[...] + jnp.concatenate([a_ref[...], b_ref[...]], -1)
        out0 = _lnorm_mx(s, lng[...], lnb[...], m64[...])
        x1 = out0 * jax.nn.sigmoid(out0 @ gw[...] + gb[...])
        x1_ref[...] = x1
        oh = (bt_ref[...] == lax.broadcasted_iota(jnp.int32, (BN, 64), 1))
        oh = oh.astype(_F32)
        xa = jnp.concatenate([x1, jnp.ones((BN, 64), _F32)], -1)
        psum = lax.dot_general(oh, xa, (((0,), (0,)), ((), ())),
                               preferred_element_type=_F32)

        @pl.when(pl.program_id(0) == 0)
        def _():
            acc_ref[...] = jnp.zeros_like(acc_ref)

        acc_ref[...] += psum

    return pl.pallas_call(
        body, grid=(GRID_N,),
        in_specs=[_bspec((BN, 64), _EMAP), _bspec((BN, 32), _EMAP),
                  _bspec((BN, 32), _EMAP),
                  pl.BlockSpec((BN, 1), _EMAP)] + _wspecs(ws),
        out_specs=[_bspec((BN, 64), _EMAP), _bspec((64, 128), _WMAP0)],
        out_shape=[jax.ShapeDtypeStruct((N_PAD, 64), _F32),
                   jax.ShapeDtypeStruct((64, 128), _F32)],
        compiler_params=_ARB,
    )(x, ma, mb, batchp, *ws)


def _tc_ncc(x1, batchp, accs, vw):
    ws = [vw["a0w"], vw["a0b"], vw["alng"], vw["alnb"], vw["a1w"], vw["a1b"],
          vw["normg"], vw["normb"], vw["b0w"], vw["b0b"], vw["blng"],
          vw["blnb"]]

    def body(x_ref, bt_ref, acc_ref, *rest):
        (a0w, a0b, alng, alnb, a1w, a1b, normg, normb,
         b0w, b0b, blng, blnb) = (r[...] for r in rest[:12])
        o_ref = rest[12]
        acc = acc_ref[...]
        sums = acc[:, 0:64]
        cnt = acc[:, 64:128]
        mean = sums / jnp.maximum(cnt, 1.0)
        t = _lnorm(_gelu(mean @ a0w + a0b), alng, alnb) @ a1w + a1b
        vnn = _lnorm(t, normg, normb)
        brow = _lnorm(_gelu(vnn @ b0w + b0b), blng, blnb)
        oh = (bt_ref[...] == lax.broadcasted_iota(jnp.int32, (BN, 64), 1))
        o_ref[...] = x_ref[...] + oh.astype(_F32) @ brow

    return pl.pallas_call(
        body, grid=(GRID_N,),
        in_specs=[_bspec((BN, 64), _EMAP), pl.BlockSpec((BN, 1), _EMAP),
                  _bspec((64, 128), _WMAP0)] + _wspecs(ws),
        out_specs=_bspec((BN, 64), _EMAP),
        out_shape=jax.ShapeDtypeStruct((N_PAD, 64), _F32),
        compiler_params=_ARB,
    )(x1, batchp, accs, *ws)


def _tc_final(hn, fa, fb, ow):
    ws = [ow["w"], ow["b"]]

    def body(h_ref, a_ref, b_ref, w_ref, bb_ref, o_ref):
        f = jnp.concatenate([a_ref[...], b_ref[...]], -1)
        h = h_ref[...] + 0.1 * f
        o_ref[...] = h @ w_ref[...] + bb_ref[...]

    return pl.pallas_call(
        body, grid=(GRID_N,),
        in_specs=[_bspec((BN, 64), _EMAP), _bspec((BN, 32), _EMAP),
                  _bspec((BN, 32), _EMAP)] + _wspecs(ws),
        out_specs=_bspec((BN, 64), _EMAP),
        out_shape=jax.ShapeDtypeStruct((N_PAD, 64), _F32),
        compiler_params=_ARB,
    )(hn, fa, fb, *ws)


def _padr(w, rows):
    return jnp.pad(w, ((0, rows - w.shape[0]), (0, 0)))


def _prep_rbf(p):
    return {"c": p["centers"][None, :],
            "inv_w": 1.0 / (jnp.abs(p["widths"]) + 1e-5)[None, :],
            "p0w": p["p0"]["w"], "p0b": p["p0"]["b"][None, :],
            "p1w": p["p1"]["w"], "p1b": p["p1"]["b"][None, :]}


def _prep_egnn(p):
    w = p["e0"]["w"]
    n0 = p["n0"]["w"]
    return {
        "whr": _padr(w[0:64], 80), "whc": _padr(w[64:128], 80),
        "wd": w[128:129], "wea": _padr(w[129:142], 16), "wrb": w[142:206],
        "e0b": p["e0"]["b"][None, :],
        "elng": p["eln"]["g"][None, :], "elnb": p["eln"]["b"][None, :],
        "e1w": p["e1"]["w"], "e1b": p["e1"]["b"][None, :],
        "attw": jnp.tile(p["att"]["w"], (1, 8)),
        "attb": jnp.tile(p["att"]["b"], 8)[None, :],
        "c0w": p["c0"]["w"], "c0b": p["c0"]["b"][None, :],
        "c1w": jnp.tile(p["c1"]["w"], (1, 8)),
        "wnh": n0[0:64], "wna": n0[64:96], "wnb": n0[96:128],
        "n0b": p["n0"]["b"][None, :],
        "nlng": p["nln"]["g"][None, :], "nlnb": p["nln"]["b"][None, :],
        "n1w": p["n1"]["w"], "n1b": p["n1"]["b"][None, :],
        "normg": p["norm"]["g"][None, :], "normb": p["norm"]["b"][None, :],
        "f0w": p["f0"]["w"], "f0b": p["f0"]["b"][None, :],
        "f1w": p["f1"]["w"], "f1b": p["f1"]["b"][None, :],
    }


def _prep_cf(p):
    r = _prep_rbf(p["rbf"])
    return {"rc": r["c"], "riw": r["inv_w"], "rp0w": r["p0w"],
            "rp0b": r["p0b"], "rp1w": r["p1w"], "rp1b": r["p1b"],
            "w0w": p["w0"]["w"], "w0b": p["w0"]["b"][None, :],
            "w1w": p["w1"]["w"], "w1b": p["w1"]["b"][None, :],
            "w2w": p["w2"]["w"], "w2b": p["w2"]["b"][None, :],
            "npw": p["np"]["w"], "npb": p["np"]["b"][None, :],
            "lng": p["ln"]["g"][None, :], "lnb": p["ln"]["b"][None, :],
            "gw": p["gate"]["w"], "gb": p["gate"]["b"][None, :]}


def _prep_vn(p):
    return {"a0w": p["a0"]["w"], "a0b": p["a0"]["b"][None, :],
            "alng": p["aln"]["g"][None, :], "alnb": p["aln"]["b"][None, :],
            "a1w": p["a1"]["w"], "a1b": p["a1"]["b"][None, :],
            "normg": p["norm"]["g"][None, :], "normb": p["norm"]["b"][None, :],
            "b0w": p["b0"]["w"], "b0b": p["b0"]["b"][None, :],
            "blng": p["bln"]["g"][None, :], "blnb": p["bln"]["b"][None, :]}


def kernel(x, pos, edge_index, edge_attr, batch, params):
    row = edge_index[0]
    col = edge_index[1]
    rowp = jnp.pad(row, (0, E_PAD - E), constant_values=N).reshape(-1, EBLK)
    colp = jnp.pad(col, (0, E_PAD - E), constant_values=N).reshape(-1, EBLK)
    ea16 = jnp.pad(edge_attr, ((0, E_PAD - E), (0, 16 - BOND)))
    xp = jnp.pad(x, ((0, N_PAD - N), (0, 48 - IN_DIM)))
    pos16 = jnp.pad(pos, ((0, N_PAD - N), (0, 13)))
    batchp = jnp.pad(batch[:, None], ((0, N_PAD - N), (0, 0)),
                     constant_values=NGRAPH)

    erw = _prep_rbf(params["edge_rbf"])
    iw = {"w": _padr(params["in0"]["w"], 48), "b": params["in0"]["b"][None, :],
          "g": params["inln"]["g"][None, :], "bb": params["inln"]["b"][None, :]}
    b0 = _prep_egnn(params["blocks"][0]["egnn"])
    b1 = _prep_egnn(params["blocks"][1]["egnn"])
    cw = _prep_cf(params["blocks"][0]["cf"])
    vw = _prep_vn(params["blocks"][0]["vn"])
    ow = {"w": params["out"]["w"], "b": params["out"]["b"][None, :]}
    m128 = _mones(128)
    m64 = _mones(64)
    sp = {"s16": jnp.ones((16, 16), _F32),
          "o16_64": jnp.full((16, 64), 1.0 / 16, _F32),
          "o16_128": jnp.full((16, 128), 1.0 / 16, _F32),
          "o8_64": jnp.full((8, 64), 1.0 / 8, _F32),
          "o8_16": jnp.full((8, 16), 1.0 / 8, _F32)}

    h0 = _tc_input(xp, iw, m64)
    t0 = jnp.concatenate([h0, pos16], 1)

    tr, tc_ = _sc_gather(t0, [rowp, colp])
    mm0, pd0, rbfv = _tc_edge(tr, tc_, ea16, None, b0, erw, m128, sp,
                              first=True)
    agg0 = _sc_scatter(mm0, rowp, 32)
    pda0 = _sc_scatter(pd0, rowp, 16, edge_split=True)
    hn0, pos1 = _tc_node(h0, agg0[0], agg0[1], pda0[0], pda0[1], pos16, b0,
                         m128, m64)
    pr, pc = _sc_gather(pos1, [rowp, colp])
    ff0, d1 = _tc_fb(pr, pc, b0, sp, first=True)
    fagg0 = _sc_scatter(ff0, rowp, 32)
    xcf = _tc_addf(hn0, fagg0[0], fagg0[1])

    (xc,) = _sc_gather(xcf, [colp])
    cm = _tc_cf(xc, d1, cw, sp)
    cagg = _sc_scatter(cm, rowp, 32)
    x1, accs = _tc_nca(xcf, cagg[0], cagg[1], batchp, cw, m64)
    x2 = _tc_ncc(x1, batchp, accs, vw)

    t2 = jnp.concatenate([x2, pos1], 1)
    tr2, tc2 = _sc_gather(t2, [rowp, colp])
    mm1, pd1 = _tc_edge(tr2, tc2, ea16, rbfv, b1, erw, m128, sp, first=False)
    agg1 = _sc_scatter(mm1, rowp, 32)
    pda1 = _sc_scatter(pd1, rowp, 16, edge_split=True)
    hn1, pos2 = _tc_node(x2, agg1[0], agg1[1], pda1[0], pda1[1], pos1, b1,
                         m128, m64)
    pr2, pc2 = _sc_gather(pos2, [rowp, colp])
    ff1 = _tc_fb(pr2, pc2, b1, sp, first=False)
    fagg1 = _sc_scatter(ff1, rowp, 32)

    out = _tc_final(hn1, fagg1[0], fagg1[1], ow)
    return out[:N]


# speedup vs baseline: 1.1314x; 1.0250x over previous
import math
import functools

import jax
import jax.numpy as jnp
from jax import lax
from jax.experimental import pallas as pl
from jax.experimental.pallas import tpu as pltpu
from jax.experimental.pallas import tpu_sc as plsc

N = 50000
E = 800000
IN_DIM = 47
H = 64
NRBF = 64
BOND = 13
NB = 2
NGRAPH = 64
OUT = 64
CUT = 10.0

NC, NS = 2, 16
NW = NC * NS
EBLK = 128
BLK_PER_W = 196
E_PAD = NW * BLK_PER_W * EBLK
N_PAD = 51200

BE = 4096
GRID_E = E_PAD // BE
BN = 512
GRID_N = N_PAD // BN

_F32 = jnp.float32


K_G = 7


def _sc_gather(tab, idxs):
    D = tab.shape[1]
    n = len(idxs)
    ngrp = BLK_PER_W // K_G
    mesh = plsc.VectorSubcoreMesh(core_axis_name="c", subcore_axis_name="s",
                                  num_cores=NC, num_subcores=NS)
    out_type = [jax.ShapeDtypeStruct((E_PAD, D), _F32)] * n
    scratch = [pltpu.VMEM((BLK_PER_W, EBLK), jnp.int32),
               pltpu.VMEM((K_G, EBLK, D), _F32),
               pltpu.SemaphoreType.DMA,
               pltpu.SemaphoreType.DMA]

    def body(tab_ref, *rest):
        idx_refs = rest[:n]
        out_refs = rest[n:2 * n]
        idxb, rowsb, sem_g, sem_w = rest[2 * n:]
        c = lax.axis_index("c")
        s = lax.axis_index("s")
        w = s * NC + c
        blk0 = w * BLK_PER_W

        for i in range(n):
            pltpu.sync_copy(idx_refs[i].at[pl.ds(blk0, BLK_PER_W), :], idxb)

            def grp(g, carry):
                b0 = g * K_G
                gd = [pltpu.async_copy(tab_ref.at[idxb.at[b0 + k]],
                                       rowsb.at[k], sem_g)
                      for k in range(K_G)]
                for d in gd:
                    d.wait()
                wd = [pltpu.async_copy(
                    rowsb.at[k],
                    out_refs[i].at[pl.ds((blk0 + b0 + k) * EBLK, EBLK), :],
                    sem_w) for k in range(K_G)]
                for d in wd:
                    d.wait()
                return carry

            lax.fori_loop(0, ngrp, grp, 0)

    f = pl.kernel(body, out_type=out_type, mesh=mesh, scratch_types=scratch,
                  compiler_params=pltpu.CompilerParams(
                      use_tc_tiling_on_sc=False))
    return f(tab, *idxs)


def _sc_scatter(msg, idx, dh, edge_split=False):
    p = msg.shape[0]
    rps = N_PAD // NS
    rb = rps // EBLK
    nblk = E_PAD // EBLK
    if edge_split:
        bps = nblk // (NC * NS)
        ks = 7
    else:
        bps = nblk // NS
        ks = 4 if dh >= 32 else 8
    ngrp = bps // ks
    mesh = plsc.VectorSubcoreMesh(core_axis_name="c", subcore_axis_name="s",
                                  num_cores=NC, num_subcores=NS)
    out_type = jax.ShapeDtypeStruct((2, N_PAD, dh), _F32)
    scratch = [pltpu.VMEM_SHARED((N_PAD, dh), _F32),
               pltpu.VMEM((ks, EBLK), jnp.int32),
               pltpu.VMEM((ks, EBLK, dh), _F32),
               pltpu.VMEM((EBLK, dh), _F32),
               pltpu.SemaphoreType.DMA,
               pltpu.SemaphoreType.DMA]
    offs = [o for o in (0, 16, 24, 32, 48) if o + 16 <= dh]

    def body(msg_ref, idx_ref, out_ref, acc, idxb, msgb, zbuf, sem_m, sem_s):
        c = lax.axis_index("c")
        s = lax.axis_index("s")
        plane = c * (p - 1)

        def zrow(j, carry):
            for o in offs:
                zbuf[j, pl.ds(o, 16)] = jnp.zeros((16,), _F32)
            return carry

        lax.fori_loop(0, EBLK, zrow, 0)

        zd = [pltpu.async_copy(zbuf, acc.at[pl.ds(s * rps + r * EBLK, EBLK),
                                            :], sem_m) for r in range(rb)]
        for d in zd:
            d.wait()
        plsc.subcore_barrier()

        if edge_split:
            myblk0 = (c * NS + s) * bps
        else:
            myblk0 = s * bps

        def grp(g, carry):
            b0 = myblk0 + g * ks
            pltpu.sync_copy(idx_ref.at[pl.ds(b0, ks), :], idxb)
            md = [pltpu.async_copy(
                msg_ref.at[plane, pl.ds((b0 + k) * EBLK, EBLK), :],
                msgb.at[k], sem_m) for k in range(ks)]
            for d in md:
                d.wait()
            sd = [pltpu.async_copy(msgb.at[k], acc.at[idxb.at[k]], sem_s,
                                   add=True) for k in range(ks)]
            for d in sd:
                d.wait()
            return carry

        lax.fori_loop(0, ngrp, grp, 0)
        plsc.subcore_barrier()

        cd = [pltpu.async_copy(acc.at[pl.ds(s * rps + r * EBLK, EBLK), :],
                               out_ref.at[c, pl.ds(s * rps + r * EBLK, EBLK),
                                          :], sem_m) for r in range(rb)]
        for d in cd:
            d.wait()

    f = pl.kernel(body, out_type=out_type, mesh=mesh, scratch_types=scratch,
                  compiler_params=pltpu.CompilerParams(
                      use_tc_tiling_on_sc=False))
    return f(msg, idx)


def _silu(t):
    return t * jax.nn.sigmoid(t)


def _gelu(t):
    return 0.5 * t * (1.0 + lax.erf(t * 0.7071067811865476))


def _lnorm(t, g, b):
    m = jnp.mean(t, -1, keepdims=True)
    v = jnp.mean((t - m) ** 2, -1, keepdims=True)
    return (t - m) * lax.rsqrt(v + 1e-5) * g + b


def _lnorm_mx(t, g, b, mones):
    r = t - t @ mones
    v = (r * r) @ mones
    return r * lax.rsqrt(v + 1e-5) * g + b


def _rbf_tc(d, rw):
    env = 0.5 * (jnp.cos(d * (math.pi / CUT)) + 1.0)
    env = env * jnp.where(d < CUT, 1.0, 0.0)
    r = jnp.exp(-0.5 * ((d - rw["c"]) * rw["inv_w"]) ** 2)
    hh = r * env
    return _silu(hh @ rw["p0w"] + rw["p0b"]) @ rw["p1w"] + rw["p1b"]


def _bspec(shape, emap):
    return pl.BlockSpec(shape, emap)


_EMAP = lambda i: (i, 0)
_WMAP0 = lambda i: (0, 0)
_ACC3 = lambda i: (0, i, 0)

_ARB = pltpu.CompilerParams(dimension_semantics=("arbitrary",))


def _edge_specs(n80, n16, n1):
    specs = [_bspec((BE, 80), _EMAP)] * n80
    specs += [_bspec((BE, 16), _EMAP)] * n16
    specs += [_bspec((BE, 1), _EMAP)] * n1
    return specs


def _wspecs(ws):
    return [_bspec(w.shape, _WMAP0) for w in ws]


def _tc_input(xp, iw, mm64):
    ws = [iw["w"], iw["b"], iw["g"], iw["bb"], mm64]

    def body(x_ref, w_ref, b_ref, g_ref, gb_ref, m64_ref, o_ref):
        t = _gelu(x_ref[...] @ w_ref[...] + b_ref[...])
        o_ref[...] = _lnorm_mx(t, g_ref[...], gb_ref[...], m64_ref[...])

    return pl.pallas_call(
        body, grid=(GRID_N,),
        in_specs=[_bspec((BN, 48), _EMAP)] + _wspecs(ws),
        out_specs=_bspec((BN, 64), _EMAP),
        out_shape=jax.ShapeDtypeStruct((N_PAD, 64), _F32),
        compiler_params=_ARB,
    )(xp, *ws)


def _tc_edge(tr, tc_, ea16, rbf_in, ew, rw, mm128, sp, first):
    ws = [ew["whr"], ew["whc"], ew["wd"], ew["wea"], ew["wrb"], ew["e0b"],
          ew["elng"], ew["elnb"], ew["e1w"], ew["e1b"],
          ew["attw"], ew["attb"], ew["c0w"], ew["c0b"], ew["c1w"], mm128,
          sp["s16"], sp["o16_128"], sp["o8_64"], sp["o8_16"]]
    if first:
        ws += [sp["o16_64"], rw["c"], rw["inv_w"], rw["p0w"], rw["p0b"],
               rw["p1w"], rw["p1b"]]
    nw = len(ws)

    def body(tr_ref, tc_ref, ea_ref, *rest):
        if first:
            wrefs = rest[:nw]
            mm_ref, pd_ref, rb_ref = rest[nw:]
        else:
            rb_in_ref = rest[0]
            wrefs = rest[1:1 + nw]
            mm_ref, pd_ref = rest[1 + nw:]
        (whr, whc, wd, wea, wrb, e0b, elng, elnb, e1w, e1b,
         attw, attb, c0w, c0b, c1w, m128, s16, o16_128, o8_64, o8_16) = (
            r[...] for r in wrefs[:20])
        trv = tr_ref[...]
        tcv = tc_ref[...]
        diff16 = trv[:, 64:80] - tcv[:, 64:80]
        dist16 = jnp.maximum(
            jnp.sqrt((diff16 * diff16) @ s16), 1e-5)
        if first:
            o16_64, rc, riw, rp0w, rp0b, rp1w, rp1b = (
                r[...] for r in wrefs[20:])
            rwd = {"c": rc, "inv_w": riw, "p0w": rp0w, "p0b": rp0b,
                   "p1w": rp1w, "p1b": rp1b}
            rb = _rbf_tc(dist16 @ o16_64, rwd)
            rb_ref[...] = rb
        else:
            rb = rb_in_ref[...]
        t = (trv @ whr + tcv @ whc + (dist16 @ o16_128) * wd
             + ea_ref[...] @ wea + rb @ wrb + e0b)
        t = _lnorm_mx(_silu(t), elng, elnb, m128)
        m = _silu(t @ e1w + e1b)
        att = jax.nn.sigmoid(m @ attw + attb)
        matt = m * (att @ o8_64)
        cw = _silu(m @ c0w + c0b) @ c1w
        mm_ref[0, :, :] = matt[:, 0:32]
        mm_ref[1, :, :] = matt[:, 32:64]
        pd_ref[0, :, :] = diff16 * (cw @ o8_16)

    out_shape = [jax.ShapeDtypeStruct((2, E_PAD, 32), _F32),
                 jax.ShapeDtypeStruct((1, E_PAD, 16), _F32)]
    out_specs = [pl.BlockSpec((2, BE, 32), _ACC3),
                 pl.BlockSpec((1, BE, 16), _ACC3)]
    in_specs = _edge_specs(2, 1, 0)
    operands = [tr, tc_, ea16]
    if first:
        out_shape.append(jax.ShapeDtypeStruct((E_PAD, 64), _F32))
        out_specs.append(_bspec((BE, 64), _EMAP))
    else:
        in_specs += [_bspec((BE, 64), _EMAP)]
        operands.append(rbf_in)
    in_specs += _wspecs(ws)
    operands += ws

    return pl.pallas_call(
        body, grid=(GRID_E,), in_specs=in_specs, out_specs=out_specs,
        out_shape=out_shape, compiler_params=_ARB,
    )(*operands)


def _tc_node(h, agg_a, agg_b, pd_a, pd_b, pos16, ew, mm128, mm64):
    ws = [ew["wnh"], ew["wna"], ew["wnb"], ew["n0b"],
          ew["nlng"], ew["nlnb"], ew["n1w"], ew["n1b"],
          ew["normg"], ew["normb"], mm128, mm64]

    def body(h_ref, a_ref, b_ref, pda_ref, pdb_ref, p_ref, *rest):
        (wnh, wna, wnb, n0b, nlng, nlnb, n1w, n1b, normg, normb,
         m128, m64) = (r[...] for r in rest[:12])
        hn_ref, p1_ref = rest[12:]
        hv = h_ref[...]
        t = hv @ wnh + a_ref[...] @ wna + b_ref[...] @ wnb + n0b
        t = _lnorm_mx(_silu(t), nlng, nlnb, m128) @ n1w + n1b
        hn_ref[...] = _lnorm_mx(hv + t, normg, normb, m64)
        p1_ref[...] = p_ref[...] + pda_ref[...] + pdb_ref[...]

    return pl.pallas_call(
        body, grid=(GRID_N,),
        in_specs=[_bspec((BN, 64), _EMAP), _bspec((BN, 32), _EMAP),
                  _bspec((BN, 32), _EMAP), _bspec((BN, 16), _EMAP),
                  _bspec((BN, 16), _EMAP), _bspec((BN, 16), _EMAP)]
        + _wspecs(ws),
        out_specs=[_bspec((BN, 64), _EMAP), _bspec((BN, 16), _EMAP)],
        out_shape=[jax.ShapeDtypeStruct((N_PAD, 64), _F32),
                   jax.ShapeDtypeStruct((N_PAD, 16), _F32)],
        compiler_params=_ARB,
    )(h, agg_a, agg_b, pd_a, pd_b, pos16, *ws)


def _mones(d):
    return jnp.full((d, d), 1.0 / d, _F32)


def _tc_fb(pr, pc, ew, sp, first):
    ws = [ew["f0w"], ew["f0b"], ew["f1w"], ew["f1b"], sp["s16"],
          sp["o16_64"]]

    def body(pr_ref, pc_ref, w0, b0, w1, b1, s16, o16_64, *outs):
        diff = pr_ref[...] - pc_ref[...]
        dn16 = jnp.maximum(jnp.sqrt((diff * diff) @ s16[...]), 1e-5)
        fb = _silu((dn16 @ o16_64[...]) * w0[...] + b0[...]) @ w1[...] \
            + b1[...]
        outs[0][0, :, :] = fb[:, 0:32]
        outs[0][1, :, :] = fb[:, 32:64]
        if first:
            outs[1][...] = dn16

    out_shape = [jax.ShapeDtypeStruct((2, E_PAD, 32), _F32)]
    out_specs = [pl.BlockSpec((2, BE, 32), _ACC3)]
    if first:
        out_shape.append(jax.ShapeDtypeStruct((E_PAD, 16), _F32))
        out_specs.append(_bspec((BE, 16), _EMAP))

    res = pl.pallas_call(
        body, grid=(GRID_E,),
        in_specs=_edge_specs(0, 2, 0) + _wspecs(ws),
        out_specs=out_specs, out_shape=out_shape, compiler_params=_ARB,
    )(pr, pc, *ws)
    return res if first else res[0]


def _tc_addf(hn, fa, fb):
    def body(h_ref, a_ref, b_ref, o_ref):
        f = jnp.concatenate([a_ref[...], b_ref[...]], -1)
        o_ref[...] = h_ref[...] + 0.1 * f

    return pl.pallas_call(
        body, grid=(GRID_N,),
        in_specs=[_bspec((BN, 64), _EMAP), _bspec((BN, 32), _EMAP),
                  _bspec((BN, 32), _EMAP)],
        out_specs=_bspec((BN, 64), _EMAP),
        out_shape=jax.ShapeDtypeStruct((N_PAD, 64), _F32),
        compiler_params=_ARB,
    )(hn, fa, fb)


def _tc_cf(xc, d1, cw, sp):
    ws = [cw["rc"], cw["riw"], cw["rp0w"], cw["rp0b"], cw["rp1w"], cw["rp1b"],
          cw["w0w"], cw["w0b"], cw["w1w"], cw["w1b"], cw["w2w"], cw["w2b"],
          cw["npw"], cw["npb"], sp["o16_64"]]

    def body(x_ref, d_ref, *rest):
        (rc, riw, rp0w, rp0b, rp1w, rp1b,
         w0w, w0b, w1w, w1b, w2w, w2b, npw, npb, o16_64) = (
            r[...] for r in rest[:15])
        o_ref = rest[15]
        rwd = {"c": rc, "inv_w": riw, "p0w": rp0w, "p0b": rp0b,
               "p1w": rp1w, "p1b": rp1b}
        rb = _rbf_tc(d_ref[...] @ o16_64, rwd)
        wf = _silu(_silu(rb @ w0w + w0b) @ w1w + w1b) @ w2w + w2b
        msg = (x_ref[...] @ npw + npb) * wf
        o_ref[0, :, :] = msg[:, 0:32]
        o_ref[1, :, :] = msg[:, 32:64]

    return pl.pallas_call(
        body, grid=(GRID_E,),
        in_specs=[_bspec((BE, 64), _EMAP), _bspec((BE, 16), _EMAP)]
        + _wspecs(ws),
        out_specs=pl.BlockSpec((2, BE, 32), _ACC3),
        out_shape=jax.ShapeDtypeStruct((2, E_PAD, 32), _F32),
        compiler_params=_ARB,
    )(xc, d1, *ws)


def _tc_nca(x, ma, mb, batchp, cw, mm64):
    ws = [cw["lng"], cw["lnb"], cw["gw"], cw["gb"], mm64]

    def body(x_ref, a_ref, b_ref, bt_ref, lng, lnb, gw, gb, m64, x1_ref,
             acc_ref):
        s = x_ref[...] + jnp.concatenate([a_ref[...], b_ref[...]], -1)
        out0 = _lnorm_mx(s, lng[...], lnb[...], m64[...])
        x1 = out0 * jax.nn.sigmoid(out0 @ gw[...] + gb[...])
        x1_ref[...] = x1
        oh = (bt_ref[...] == lax.broadcasted_iota(jnp.int32, (BN, 64), 1))
        oh = oh.astype(_F32)
        xa = jnp.concatenate([x1, jnp.ones((BN, 64), _F32)], -1)
        psum = lax.dot_general(oh, xa, (((0,), (0,)), ((), ())),
                               preferred_element_type=_F32)

        @pl.when(pl.program_id(0) == 0)
        def _():
            acc_ref[...] = jnp.zeros_like(acc_ref)

        acc_ref[...] += psum

    return pl.pallas_call(
        body, grid=(GRID_N,),
        in_specs=[_bspec((BN, 64), _EMAP), _bspec((BN, 32), _EMAP),
                  _bspec((BN, 32), _EMAP),
                  pl.BlockSpec((BN, 1), _EMAP)] + _wspecs(ws),
        out_specs=[_bspec((BN, 64), _EMAP), _bspec((64, 128), _WMAP0)],
        out_shape=[jax.ShapeDtypeStruct((N_PAD, 64), _F32),
                   jax.ShapeDtypeStruct((64, 128), _F32)],
        compiler_params=_ARB,
    )(x, ma, mb, batchp, *ws)


def _tc_ncc(x1, batchp, accs, vw):
    ws = [vw["a0w"], vw["a0b"], vw["alng"], vw["alnb"], vw["a1w"], vw["a1b"],
          vw["normg"], vw["normb"], vw["b0w"], vw["b0b"], vw["blng"],
          vw["blnb"]]

    def body(x_ref, bt_ref, acc_ref, *rest):
        (a0w, a0b, alng, alnb, a1w, a1b, normg, normb,
         b0w, b0b, blng, blnb) = (r[...] for r in rest[:12])
        o_ref = rest[12]
        acc = acc_ref[...]
        sums = acc[:, 0:64]
        cnt = acc[:, 64:128]
        mean = sums / jnp.maximum(cnt, 1.0)
        t = _lnorm(_gelu(mean @ a0w + a0b), alng, alnb) @ a1w + a1b
        vnn = _lnorm(t, normg, normb)
        brow = _lnorm(_gelu(vnn @ b0w + b0b), blng, blnb)
        oh = (bt_ref[...] == lax.broadcasted_iota(jnp.int32, (BN, 64), 1))
        o_ref[...] = x_ref[...] + oh.astype(_F32) @ brow

    return pl.pallas_call(
        body, grid=(GRID_N,),
        in_specs=[_bspec((BN, 64), _EMAP), pl.BlockSpec((BN, 1), _EMAP),
                  _bspec((64, 128), _WMAP0)] + _wspecs(ws),
        out_specs=_bspec((BN, 64), _EMAP),
        out_shape=jax.ShapeDtypeStruct((N_PAD, 64), _F32),
        compiler_params=_ARB,
    )(x1, batchp, accs, *ws)


def _tc_final(hn, fa, fb, ow):
    ws = [ow["w"], ow["b"]]

    def body(h_ref, a_ref, b_ref, w_ref, bb_ref, o_ref):
        f = jnp.concatenate([a_ref[...], b_ref[...]], -1)
        h = h_ref[...] + 0.1 * f
        o_ref[...] = h @ w_ref[...] + bb_ref[...]

    return pl.pallas_call(
        body, grid=(GRID_N,),
        in_specs=[_bspec((BN, 64), _EMAP), _bspec((BN, 32), _EMAP),
                  _bspec((BN, 32), _EMAP)] + _wspecs(ws),
        out_specs=_bspec((BN, 64), _EMAP),
        out_shape=jax.ShapeDtypeStruct((N_PAD, 64), _F32),
        compiler_params=_ARB,
    )(hn, fa, fb, *ws)


def _padr(w, rows):
    return jnp.pad(w, ((0, rows - w.shape[0]), (0, 0)))


def _prep_rbf(p):
    return {"c": p["centers"][None, :],
            "inv_w": 1.0 / (jnp.abs(p["widths"]) + 1e-5)[None, :],
            "p0w": p["p0"]["w"], "p0b": p["p0"]["b"][None, :],
            "p1w": p["p1"]["w"], "p1b": p["p1"]["b"][None, :]}


def _prep_egnn(p):
    w = p["e0"]["w"]
    n0 = p["n0"]["w"]
    return {
        "whr": _padr(w[0:64], 80), "whc": _padr(w[64:128], 80),
        "wd": w[128:129], "wea": _padr(w[129:142], 16), "wrb": w[142:206],
        "e0b": p["e0"]["b"][None, :],
        "elng": p["eln"]["g"][None, :], "elnb": p["eln"]["b"][None, :],
        "e1w": p["e1"]["w"], "e1b": p["e1"]["b"][None, :],
        "attw": jnp.tile(p["att"]["w"], (1, 8)),
        "attb": jnp.tile(p["att"]["b"], 8)[None, :],
        "c0w": p["c0"]["w"], "c0b": p["c0"]["b"][None, :],
        "c1w": jnp.tile(p["c1"]["w"], (1, 8)),
        "wnh": n0[0:64], "wna": n0[64:96], "wnb": n0[96:128],
        "n0b": p["n0"]["b"][None, :],
        "nlng": p["nln"]["g"][None, :], "nlnb": p["nln"]["b"][None, :],
        "n1w": p["n1"]["w"], "n1b": p["n1"]["b"][None, :],
        "normg": p["norm"]["g"][None, :], "normb": p["norm"]["b"][None, :],
        "f0w": p["f0"]["w"], "f0b": p["f0"]["b"][None, :],
        "f1w": p["f1"]["w"], "f1b": p["f1"]["b"][None, :],
    }


def _prep_cf(p):
    r = _prep_rbf(p["rbf"])
    return {"rc": r["c"], "riw": r["inv_w"], "rp0w": r["p0w"],
            "rp0b": r["p0b"], "rp1w": r["p1w"], "rp1b": r["p1b"],
            "w0w": p["w0"]["w"], "w0b": p["w0"]["b"][None, :],
            "w1w": p["w1"]["w"], "w1b": p["w1"]["b"][None, :],
            "w2w": p["w2"]["w"], "w2b": p["w2"]["b"][None, :],
            "npw": p["np"]["w"], "npb": p["np"]["b"][None, :],
            "lng": p["ln"]["g"][None, :], "lnb": p["ln"]["b"][None, :],
            "gw": p["gate"]["w"], "gb": p["gate"]["b"][None, :]}


def _prep_vn(p):
    return {"a0w": p["a0"]["w"], "a0b": p["a0"]["b"][None, :],
            "alng": p["aln"]["g"][None, :], "alnb": p["aln"]["b"][None, :],
            "a1w": p["a1"]["w"], "a1b": p["a1"]["b"][None, :],
            "normg": p["norm"]["g"][None, :], "normb": p["norm"]["b"][None, :],
            "b0w": p["b0"]["w"], "b0b": p["b0"]["b"][None, :],
            "blng": p["bln"]["g"][None, :], "blnb": p["bln"]["b"][None, :]}


def kernel(x, pos, edge_index, edge_attr, batch, params):
    row = edge_index[0]
    col = edge_index[1]
    rowp = jnp.pad(row, (0, E_PAD - E), constant_values=N).reshape(-1, EBLK)
    colp = jnp.pad(col, (0, E_PAD - E), constant_values=N).reshape(-1, EBLK)
    ea16 = jnp.pad(edge_attr, ((0, E_PAD - E), (0, 16 - BOND)))
    xp = jnp.pad(x, ((0, N_PAD - N), (0, 48 - IN_DIM)))
    pos16 = jnp.pad(pos, ((0, N_PAD - N), (0, 13)))
    batchp = jnp.pad(batch[:, None], ((0, N_PAD - N), (0, 0)),
                     constant_values=NGRAPH)

    erw = _prep_rbf(params["edge_rbf"])
    iw = {"w": _padr(params["in0"]["w"], 48), "b": params["in0"]["b"][None, :],
          "g": params["inln"]["g"][None, :], "bb": params["inln"]["b"][None, :]}
    b0 = _prep_egnn(params["blocks"][0]["egnn"])
    b1 = _prep_egnn(params["blocks"][1]["egnn"])
    cw = _prep_cf(params["blocks"][0]["cf"])
    vw = _prep_vn(params["blocks"][0]["vn"])
    ow = {"w": params["out"]["w"], "b": params["out"]["b"][None, :]}
    m128 = _mones(128)
    m64 = _mones(64)
    sp = {"s16": jnp.ones((16, 16), _F32),
          "o16_64": jnp.full((16, 64), 1.0 / 16, _F32),
          "o16_128": jnp.full((16, 128), 1.0 / 16, _F32),
          "o8_64": jnp.full((8, 64), 1.0 / 8, _F32),
          "o8_16": jnp.full((8, 16), 1.0 / 8, _F32)}

    h0 = _tc_input(xp, iw, m64)
    t0 = jnp.concatenate([h0, pos16], 1)

    tr, tc_ = _sc_gather(t0, [rowp, colp])
    mm0, pd0, rbfv = _tc_edge(tr, tc_, ea16, None, b0, erw, m128, sp,
                              first=True)
    agg0 = _sc_scatter(mm0, rowp, 32)
    pda0 = _sc_scatter(pd0, rowp, 16, edge_split=True)
    hn0, pos1 = _tc_node(h0, agg0[0], agg0[1], pda0[0], pda0[1], pos16, b0,
                         m128, m64)
    pr, pc = _sc_gather(pos1, [rowp, colp])
    ff0, d1 = _tc_fb(pr, pc, b0, sp, first=True)
    fagg0 = _sc_scatter(ff0, rowp, 32)
    xcf = _tc_addf(hn0, fagg0[0], fagg0[1])

    (xc,) = _sc_gather(xcf, [colp])
    cm = _tc_cf(xc, d1, cw, sp)
    cagg = _sc_scatter(cm, rowp, 32)
    x1, accs = _tc_nca(xcf, cagg[0], cagg[1], batchp, cw, m64)
    x2 = _tc_ncc(x1, batchp, accs, vw)

    t2 = jnp.concatenate([x2, pos1], 1)
    tr2, tc2 = _sc_gather(t2, [rowp, colp])
    mm1, pd1 = _tc_edge(tr2, tc2, ea16, rbfv, b1, erw, m128, sp, first=False)
    agg1 = _sc_scatter(mm1, rowp, 32)
    pda1 = _sc_scatter(pd1, rowp, 16, edge_split=True)
    hn1, pos2 = _tc_node(x2, agg1[0], agg1[1], pda1[0], pda1[1], pos1, b1,
                         m128, m64)
    pr2, pc2 = _sc_gather(pos2, [rowp, colp])
    ff1 = _tc_fb(pr2, pc2, b1, sp, first=False)
    fagg1 = _sc_scatter(ff1, rowp, 32)

    out = _tc_final(hn1, fagg1[0], fagg1[1], ow)
    return out[:N]
